# Initial kernel scaffold; baseline (speedup 1.0000x reference)
import jax
import jax.numpy as jnp
from jax.experimental import pallas as pl


def kernel(x, edge_index, Wemb, bemb, We1, Wg1, asrc1, adst1, bg1, Wf1, bf1, g1, bn1, We2, Wg2, asrc2, adst2, bg2, Wf2, bf2, g2, bn2, Wr1, br1, Wr2, br2):
    raise NotImplementedError("write your pallas kernel here")


# speedup vs baseline: 17.3341x; 17.3341x over previous
import functools

import jax
import jax.numpy as jnp
from jax import lax
from jax.experimental import pallas as pl
from jax.experimental.pallas import tpu as pltpu
from jax.experimental.pallas import tpu_sc as plsc

N = 10000
D = 256
H = 4
C = 64
N_PAD = 10240
DUMMY = N
B = 128
NSC = 2
NTILE = 16
E8P = 161792
EGP = 172032
NC8 = E8P // (NTILE * B)
NCG = EGP // (NTILE * B)
RPT = N_PAD // NTILE
BM = 256
NBLK = N_PAD // BM

f32 = jnp.float32
i32 = jnp.int32


def _take16(vec, idx16):
    dnums = lax.GatherDimensionNumbers(
        offset_dims=(), collapsed_slice_dims=(0,), start_index_map=(0,))
    return lax.gather(vec, idx16[:, None], dnums, slice_sizes=(1,),
                      mode=lax.GatherScatterMode.PROMISE_IN_BOUNDS)


_MESH = plsc.VectorSubcoreMesh(
    core_axis_name="c", subcore_axis_name="s",
    num_cores=NSC, num_subcores=NTILE)

_SC_PARAMS = pltpu.CompilerParams(needs_layout_passes=False)

def _fill(ref, rows, width, scalar):
    value = jnp.full((16,), scalar, f32)
    for r in range(rows):
        for k in range(width // 16):
            if rows == 1:
                ref[pl.ds(k * 16, 16)] = value
            else:
                ref[r, pl.ds(k * 16, 16)] = value


@functools.partial(
    pl.kernel,
    out_type=jax.ShapeDtypeStruct((N_PAD,), f32),
    mesh=_MESH,
    compiler_params=_SC_PARAMS,
    scratch_types=[
        pltpu.VMEM((B,), i32),
        pltpu.VMEM((B,), f32),
        pltpu.VMEM((RPT,), f32),
        pltpu.VMEM_SHARED((N_PAD,), f32),
    ])
def _deg_kernel(gi_hbm, out_hbm, gi_c, ones_v, z_v, acc_sh):
    c = lax.axis_index("c")
    s = lax.axis_index("s")
    base = s * RPT
    _fill(ones_v, 1, B, 1.0)
    _fill(z_v, 1, RPT, 0.0)
    pltpu.sync_copy(z_v, acc_sh.at[pl.ds(base, RPT)])
    plsc.subcore_barrier()

    def chunk(j, carry):
        pltpu.sync_copy(gi_hbm.at[s, j], gi_c)
        pltpu.sync_copy(ones_v, acc_sh.at[gi_c], add=True)
        return carry

    lax.fori_loop(0, NC8, chunk, 0)
    plsc.subcore_barrier()

    @pl.when(c == 0)
    def _():
        pltpu.sync_copy(acc_sh.at[pl.ds(base, RPT)], out_hbm.at[pl.ds(base, RPT)])


@functools.partial(
    pl.kernel,
    out_type=[
        jax.ShapeDtypeStruct((NSC * N_PAD, 128), f32),
        jax.ShapeDtypeStruct((NSC * N_PAD, 128), f32),
        jax.ShapeDtypeStruct((N_PAD * H,), f32),
    ],
    mesh=_MESH,
    compiler_params=_SC_PARAMS,
    scratch_types=[
        pltpu.VMEM((2, B), i32),
        pltpu.VMEM((B,), i32),
        pltpu.VMEM((B, 128), f32),
        pltpu.VMEM((32, 128), f32),
        pltpu.VMEM((H, B), i32),
        pltpu.VMEM((H, B), i32),
        pltpu.VMEM((H, B), f32),
        pltpu.VMEM((H, B), f32),
        pltpu.VMEM((H, B), f32),
        pltpu.VMEM((1280,), f32),
        pltpu.VMEM_SHARED((N_PAD, 128), f32),
        pltpu.VMEM_SHARED((N_PAD * H,), f32),
    ])
def _layer_sc(e8i_hbm, gati_hbm, he_hbm, xl_hbm, als_hbm, ald_hbm,
              e8out_hbm, gatout_hbm, den_hbm,
              gs_c, gio_v, rows_v, z128_v, sgi_v, dgi_v, als_g, ald_g,
              exv_v, z1_v, acc_sh, den_sh):
    c = lax.axis_index("c")
    s = lax.axis_index("s")
    base = s * RPT
    dbase = s * (RPT * H)
    off = jnp.full((16,), c * N_PAD, i32)
    c2 = 2 * c
    _fill(z128_v, 32, 128, 0.0)
    _fill(z1_v, 1, 1280, 0.0)

    def zero_acc():
        for r in range(RPT // 32):
            pltpu.sync_copy(z128_v, acc_sh.at[pl.ds(base + r * 32, 32)])

    def dump_acc(out_hbm):
        pltpu.sync_copy(acc_sh.at[pl.ds(base, RPT)],
                        out_hbm.at[pl.ds(c * N_PAD + base, RPT)])

    zero_acc()
    plsc.subcore_barrier()

    def chunk8(j, carry):
        pltpu.sync_copy(e8i_hbm.at[s, j], gs_c)
        for g in range(B // 16):
            sl = pl.ds(g * 16, 16)
            gio_v[sl] = gs_c[0, sl] + off
        pltpu.sync_copy(he_hbm.at[gio_v], rows_v)
        pltpu.sync_copy(rows_v, acc_sh.at[gs_c.at[1]], add=True)
        return carry

    lax.fori_loop(0, NC8, chunk8, 0)
    plsc.subcore_barrier()
    dump_acc(e8out_hbm)
    zero_acc()
    for r in range(RPT * H // 1280):
        pltpu.sync_copy(z1_v, den_sh.at[pl.ds(dbase + r * 1280, 1280)])
    plsc.subcore_barrier()

    def chunkg(j, carry):
        pltpu.sync_copy(gati_hbm.at[s, j], gs_c)
        for g in range(B // 16):
            sl = pl.ds(g * 16, 16)
            sv = gs_c[0, sl]
            gio_v[sl] = sv + off
            dv = gs_c[1, sl]
            sv4 = sv * H
            dv4 = dv * H
            for h in range(H):
                sgi_v[h, sl] = sv4 + h
                dgi_v[h, sl] = dv4 + h
        for h in range(H):
            pltpu.sync_copy(als_hbm.at[sgi_v.at[h]], als_g.at[h])
            pltpu.sync_copy(ald_hbm.at[dgi_v.at[h]], ald_g.at[h])
        for g in range(B // 16):
            sl = pl.ds(g * 16, 16)
            for h in range(H):
                a = als_g[h, sl] + ald_g[h, sl]
                a = jnp.where(a > 0.0, a, 0.2 * a)
                exv_v[h, sl] = jnp.exp(a)
        pltpu.sync_copy(xl_hbm.at[gio_v], rows_v)
        for h in range(H):
            pltpu.sync_copy(exv_v.at[h], den_sh.at[dgi_v.at[h]], add=True)
        for g in range(B // 16):
            sl = pl.ds(g * 16, 16)
            s0 = exv_v[c2, sl]
            s1 = exv_v[c2 + 1, sl]

            def edge(b, cc):
                m0 = _take16(s0, jnp.full((16,), b, i32))
                m1 = _take16(s1, jnp.full((16,), b, i32))
                for k in range(4):
                    ksl = pl.ds(k * 16, 16)
                    rows_v[g * 16 + b, ksl] = rows_v[g * 16 + b, ksl] * m0
                for k in range(4, 8):
                    ksl = pl.ds(k * 16, 16)
                    rows_v[g * 16 + b, ksl] = rows_v[g * 16 + b, ksl] * m1
                return cc

            lax.fori_loop(0, 16, edge, 0)
        pltpu.sync_copy(rows_v, acc_sh.at[gs_c.at[1]], add=True)
        return carry

    lax.fori_loop(0, NCG, chunkg, 0)
    plsc.subcore_barrier()
    dump_acc(gatout_hbm)

    @pl.when(c == 0)
    def _():
        pltpu.sync_copy(den_sh.at[pl.ds(dbase, RPT * H)],
                        den_hbm.at[pl.ds(dbase, RPT * H)])


def _row_to_col(row):
    m = jnp.broadcast_to(row, (BM, BM))
    ri = lax.broadcasted_iota(i32, (BM, BM), 0)
    ci = lax.broadcasted_iota(i32, (BM, BM), 1)
    return jnp.sum(jnp.where(ri == ci, m, 0.0), axis=1, keepdims=True)


def _dinv_col(deg_row):
    deg = _row_to_col(deg_row)
    return jnp.where(deg > 0.0, lax.rsqrt(jnp.maximum(deg, 1e-12)), 0.0)


def _prep_body(with_emb):
    def body(*refs):
        if with_emb:
            (x_ref, wembT_ref, bemb_ref, wg_ref, asrc_ref, adst_ref, deg_ref,
             h_ref, hecat_ref, xlcat_ref, als_ref, ald_ref) = refs
            h = jnp.dot(x_ref[...], wembT_ref[...],
                        preferred_element_type=f32) + bemb_ref[...]
            h_ref[...] = h
        else:
            (x_ref, wg_ref, asrc_ref, adst_ref, deg_ref,
             hecat_ref, xlcat_ref, als_ref, ald_ref) = refs
            h = x_ref[...]
        dinv = _dinv_col(deg_ref[0])
        he = h * dinv
        hecat_ref[0] = he[:, :128]
        hecat_ref[1] = he[:, 128:]
        xl = jnp.dot(h, wg_ref[...], preferred_element_type=f32)
        xlcat_ref[0] = xl[:, :128]
        xlcat_ref[1] = xl[:, 128:]
        als_ref[...] = jnp.concatenate(
            [(xl[:, 64 * k:64 * (k + 1)] * asrc_ref[:, 64 * k:64 * (k + 1)]
              ).sum(-1, keepdims=True) for k in range(4)], axis=1)
        ald_ref[...] = jnp.concatenate(
            [(xl[:, 64 * k:64 * (k + 1)] * adst_ref[:, 64 * k:64 * (k + 1)]
              ).sum(-1, keepdims=True) for k in range(4)], axis=1)
    return body


def _wspec(shape):
    return pl.BlockSpec(shape, lambda i: (0,) * len(shape))


_PREP_OUTS = [
    jax.ShapeDtypeStruct((2, N_PAD, 128), f32),
    jax.ShapeDtypeStruct((2, N_PAD, 128), f32),
    jax.ShapeDtypeStruct((N_PAD, H), f32),
    jax.ShapeDtypeStruct((N_PAD, H), f32),
]
_PREP_OUT_SPECS = [
    pl.BlockSpec((2, BM, 128), lambda i: (0, i, 0)),
    pl.BlockSpec((2, BM, 128), lambda i: (0, i, 0)),
    pl.BlockSpec((BM, H), lambda i: (i, 0)),
    pl.BlockSpec((BM, H), lambda i: (i, 0)),
]

_prep_emb = pl.pallas_call(
    _prep_body(True),
    grid=(NBLK,),
    in_specs=[
        pl.BlockSpec((BM, D), lambda i: (i, 0)),
        _wspec((D, D)),
        _wspec((1, D)),
        _wspec((D, D)),
        _wspec((1, D)),
        _wspec((1, D)),
        pl.BlockSpec((1, 1, BM), lambda i: (i, 0, 0)),
    ],
    out_specs=[pl.BlockSpec((BM, D), lambda i: (i, 0))] + _PREP_OUT_SPECS,
    out_shape=[jax.ShapeDtypeStruct((N_PAD, D), f32)] + _PREP_OUTS,
)

_prep_noemb = pl.pallas_call(
    _prep_body(False),
    grid=(NBLK,),
    in_specs=[
        pl.BlockSpec((BM, D), lambda i: (i, 0)),
        _wspec((D, D)),
        _wspec((1, D)),
        _wspec((1, D)),
        pl.BlockSpec((1, 1, BM), lambda i: (i, 0, 0)),
    ],
    out_specs=_PREP_OUT_SPECS,
    out_shape=_PREP_OUTS,
)


def _post_body(e8_ref, gat_ref, den_ref, deg_ref, weT_ref, wfT_ref, bf_ref,
               bg_ref, res_ref, g_ref, bn_ref, out_ref):
    dinv = _dinv_col(deg_ref[0])
    agg = jnp.concatenate([e8_ref[0], e8_ref[1]], axis=1) * dinv
    xe = jnp.dot(agg, weT_ref[...], preferred_element_type=f32)
    gato = jnp.concatenate([gat_ref[0], gat_ref[1]], axis=1)
    xg = jnp.concatenate(
        [gato[:, 64 * k:64 * (k + 1)] / (den_ref[:, k:k + 1] + 1e-16)
         for k in range(4)], axis=1) + bg_ref[...]
    wfT = wfT_ref[...]
    h2 = (jnp.dot(xe, wfT[:D], preferred_element_type=f32)
          + jnp.dot(xg, wfT[D:], preferred_element_type=f32)
          + bf_ref[...] + res_ref[...])
    m = jnp.mean(h2, axis=-1, keepdims=True)
    v = jnp.mean((h2 - m) ** 2, axis=-1, keepdims=True)
    out_ref[...] = jnp.maximum(
        (h2 - m) / jnp.sqrt(v + 1e-5) * g_ref[...] + bn_ref[...], 0.0)


_post = pl.pallas_call(
    _post_body,
    grid=(NBLK,),
    in_specs=[
        pl.BlockSpec((2, BM, 128), lambda i: (0, i, 0)),
        pl.BlockSpec((2, BM, 128), lambda i: (0, i, 0)),
        pl.BlockSpec((BM, H), lambda i: (i, 0)),
        pl.BlockSpec((1, 1, BM), lambda i: (i, 0, 0)),
        _wspec((D, D)),
        _wspec((2 * D, D)),
        _wspec((1, D)),
        _wspec((1, D)),
        pl.BlockSpec((BM, D), lambda i: (i, 0)),
        _wspec((1, D)),
        _wspec((1, D)),
    ],
    out_specs=pl.BlockSpec((BM, D), lambda i: (i, 0)),
    out_shape=jax.ShapeDtypeStruct((N_PAD, D), f32),
)


def _readout_body(h_ref, wr1T_ref, br1_ref, wr2_ref, br2_ref, out_ref):
    r = jnp.maximum(
        jnp.dot(h_ref[...], wr1T_ref[...], preferred_element_type=f32)
        + br1_ref[...], 0.0)
    o = (r * wr2_ref[...]).sum(-1, keepdims=True) + br2_ref[...]
    out_ref[...] = 1.0 / (1.0 + jnp.exp(-o))


_readout = pl.pallas_call(
    _readout_body,
    grid=(NBLK,),
    in_specs=[
        pl.BlockSpec((BM, D), lambda i: (i, 0)),
        _wspec((D, D // 2)),
        _wspec((1, D // 2)),
        _wspec((1, D // 2)),
        _wspec((1, 1)),
    ],
    out_specs=pl.BlockSpec((BM, 1), lambda i: (i, 0)),
    out_shape=jax.ShapeDtypeStruct((N_PAD, 1), f32),
)


def _pad_idx(idx, tot):
    return jnp.concatenate(
        [idx, jnp.full((tot - idx.shape[0],), DUMMY, idx.dtype)])


def kernel(x, edge_index, Wemb, bemb, We1, Wg1, asrc1, adst1, bg1, Wf1, bf1,
           g1, bn1, We2, Wg2, asrc2, adst2, bg2, Wf2, bf2, g2, bn2, Wr1, br1,
           Wr2, br2):
    ei = edge_index.astype(i32)
    row, col = ei[0], ei[1]
    loop = jnp.arange(N, dtype=i32)
    g8 = _pad_idx(col, E8P).reshape(NTILE, NC8, 1, B)
    s8 = _pad_idx(row, E8P).reshape(NTILE, NC8, 1, B)
    e8i = jnp.concatenate([g8, s8], axis=2)
    sg = _pad_idx(jnp.concatenate([row, loop]), EGP).reshape(NTILE, NCG, 1, B)
    dg = _pad_idx(jnp.concatenate([col, loop]), EGP).reshape(NTILE, NCG, 1, B)
    gati = jnp.concatenate([sg, dg], axis=2)

    x_p = jnp.pad(x, ((0, N_PAD - N), (0, 0)))

    r2 = lambda a: a.reshape(1, -1)

    deg = _deg_kernel(g8.reshape(NTILE, NC8, B)).reshape(NBLK, 1, BM)

    def layer(h_or_x, Wemb_args, Wg, asrc, adst, bg, We, Wf, bf, g, bn):
        if Wemb_args is not None:
            WembT, bemb_ = Wemb_args
            h, hecat, xlcat, als, ald = _prep_emb(
                h_or_x, WembT, r2(bemb_), Wg, r2(asrc), r2(adst), deg)
        else:
            h = h_or_x
            hecat, xlcat, als, ald = _prep_noemb(
                h, Wg, r2(asrc), r2(adst), deg)
        hecat_f = hecat.reshape(NSC * N_PAD, 128)
        xlcat_f = xlcat.reshape(NSC * N_PAD, 128)
        e8o, gato, den = _layer_sc(e8i, gati, hecat_f, xlcat_f,
                                   als.reshape(N_PAD * H),
                                   ald.reshape(N_PAD * H))
        return _post(e8o.reshape(NSC, N_PAD, 128),
                     gato.reshape(NSC, N_PAD, 128),
                     den.reshape(N_PAD, H), deg, We.T, Wf.T, r2(bf), r2(bg),
                     h, r2(g), r2(bn))

    h1 = layer(x_p, (Wemb.T, bemb), Wg1, asrc1.reshape(1, D),
               adst1.reshape(1, D), bg1, We1, Wf1, bf1, g1, bn1)
    h2 = layer(h1, None, Wg2, asrc2.reshape(1, D),
               adst2.reshape(1, D), bg2, We2, Wf2, bf2, g2, bn2)
    out = _readout(h2, Wr1.T, r2(br1), Wr2, br2.reshape(1, 1))
    return out[:N]


# speedup vs baseline: 28.2966x; 1.6324x over previous
import functools

import jax
import jax.numpy as jnp
from jax import lax
from jax.experimental import pallas as pl
from jax.experimental.pallas import tpu as pltpu
from jax.experimental.pallas import tpu_sc as plsc

N = 10000
D = 256
H = 4
C = 64
N_PAD = 10240
DUMMY = N
B = 128
NSC = 2
NTILE = 16
E8P = 163840
EGP = 172032
NC8 = E8P // (NTILE * B)
NCG = EGP // (NTILE * B)
RPT = N_PAD // NTILE
BM = 256
NBLK = N_PAD // BM

f32 = jnp.float32
i32 = jnp.int32


def _take16(vec, idx16):
    dnums = lax.GatherDimensionNumbers(
        offset_dims=(), collapsed_slice_dims=(0,), start_index_map=(0,))
    return lax.gather(vec, idx16[:, None], dnums, slice_sizes=(1,),
                      mode=lax.GatherScatterMode.PROMISE_IN_BOUNDS)


_MESH = plsc.VectorSubcoreMesh(
    core_axis_name="c", subcore_axis_name="s",
    num_cores=NSC, num_subcores=NTILE)

_SC_PARAMS = pltpu.CompilerParams(needs_layout_passes=False)

def _fill(ref, rows, width, scalar):
    value = jnp.full((16,), scalar, f32)
    for r in range(rows):
        for k in range(width // 16):
            if rows == 1:
                ref[pl.ds(k * 16, 16)] = value
            else:
                ref[r, pl.ds(k * 16, 16)] = value


@functools.partial(
    pl.kernel,
    out_type=jax.ShapeDtypeStruct((N_PAD,), f32),
    mesh=_MESH,
    compiler_params=_SC_PARAMS,
    scratch_types=[
        pltpu.VMEM((B,), i32),
        pltpu.VMEM((B,), f32),
        pltpu.VMEM((RPT,), f32),
        pltpu.VMEM_SHARED((N_PAD,), f32),
    ])
def _deg_kernel(gi_hbm, out_hbm, gi_c, ones_v, z_v, acc_sh):
    c = lax.axis_index("c")
    s = lax.axis_index("s")
    base = s * RPT
    _fill(ones_v, 1, B, 1.0)
    _fill(z_v, 1, RPT, 0.0)
    pltpu.sync_copy(z_v, acc_sh.at[pl.ds(base, RPT)])
    plsc.subcore_barrier()

    def chunk(j, carry):
        pltpu.sync_copy(gi_hbm.at[s, j], gi_c)
        pltpu.sync_copy(ones_v, acc_sh.at[gi_c], add=True)
        return carry

    lax.fori_loop(0, NC8, chunk, 0)
    plsc.subcore_barrier()

    @pl.when(c == 0)
    def _():
        pltpu.sync_copy(acc_sh.at[pl.ds(base, RPT)], out_hbm.at[pl.ds(base, RPT)])


@functools.partial(
    pl.kernel,
    out_type=[
        jax.ShapeDtypeStruct((NSC * N_PAD, 128), f32),
        jax.ShapeDtypeStruct((NSC * N_PAD, 128), f32),
        jax.ShapeDtypeStruct((N_PAD * H,), f32),
    ],
    mesh=_MESH,
    compiler_params=_SC_PARAMS,
    scratch_types=[
        pltpu.VMEM((2, 2, B), i32),
        pltpu.VMEM((2, B), i32),
        pltpu.VMEM((2, B, 128), f32),
        pltpu.VMEM((32, 128), f32),
        pltpu.VMEM((2, H, B), i32),
        pltpu.VMEM((2, H, B), i32),
        pltpu.VMEM((2, H, B), f32),
        pltpu.VMEM((2, H, B), f32),
        pltpu.VMEM((2, H, B), f32),
        pltpu.VMEM((1280,), f32),
        pltpu.VMEM_SHARED((N_PAD, 128), f32),
        pltpu.VMEM_SHARED((N_PAD * H,), f32),
        pltpu.SemaphoreType.DMA,
        pltpu.SemaphoreType.DMA,
        pltpu.SemaphoreType.DMA,
        pltpu.SemaphoreType.DMA,
        pltpu.SemaphoreType.DMA,
        pltpu.SemaphoreType.DMA,
        pltpu.SemaphoreType.DMA,
        pltpu.SemaphoreType.DMA,
        pltpu.SemaphoreType.DMA,
        pltpu.SemaphoreType.DMA,
    ])
def _layer_sc(e8i_hbm, gati_hbm, he_hbm, xl_hbm, als_hbm, ald_hbm,
              e8out_hbm, gatout_hbm, den_hbm,
              gs_c, gio_v, rows_v, z128_v, sgi_v, dgi_v, als_g, ald_g,
              exv_v, z1_v, acc_sh, den_sh,
              im0, im1, al0, al1, fg0, fg1, dn0, dn1, st0, st1):
    c = lax.axis_index("c")
    s = lax.axis_index("s")
    base = s * RPT
    dbase = s * (RPT * H)
    off = jnp.full((16,), c * N_PAD, i32)
    c2 = 2 * c
    im = (im0, im1)
    al = (al0, al1)
    fg = (fg0, fg1)
    dn = (dn0, dn1)
    st = (st0, st1)
    _fill(z128_v, 32, 128, 0.0)
    _fill(z1_v, 1, 1280, 0.0)

    def zero_acc():
        for r in range(RPT // 32):
            pltpu.sync_copy(z128_v, acc_sh.at[pl.ds(base + r * 32, 32)])

    def dump_acc(out_hbm):
        pltpu.sync_copy(acc_sh.at[pl.ds(base, RPT)],
                        out_hbm.at[pl.ds(c * N_PAD + base, RPT)])

    def wait_idx(idx_hbm, j, b):
        pltpu.make_async_copy(idx_hbm.at[s, j], gs_c.at[b], im[b]).wait()

    def issue_idx(idx_hbm, j, b):
        pltpu.async_copy(idx_hbm.at[s, j], gs_c.at[b], im[b])

    def wait_scatter(b):
        pltpu.make_async_copy(rows_v.at[b], acc_sh.at[gs_c.at[b, 1]],
                              st[b]).wait()

    zero_acc()
    plsc.subcore_barrier()
    issue_idx(e8i_hbm, 0, 0)

    def chunk8(jj, carry):
        for b in range(2):
            j = 2 * jj + b
            b1 = 1 - b
            wait_idx(e8i_hbm, j, b)
            for g in range(B // 16):
                sl = pl.ds(g * 16, 16)
                gio_v[b, sl] = gs_c[b, 0, sl] + off
            pltpu.async_copy(he_hbm.at[gio_v.at[b]], rows_v.at[b], fg[b])

            @pl.when(j >= 1)
            def _():
                wait_scatter(b1)

            @pl.when(j + 1 < NC8)
            def _():
                issue_idx(e8i_hbm, j + 1, b1)

            pltpu.make_async_copy(he_hbm.at[gio_v.at[b]], rows_v.at[b],
                                  fg[b]).wait()
            pltpu.async_copy(rows_v.at[b], acc_sh.at[gs_c.at[b, 1]], st[b],
                             add=True)
        return carry

    lax.fori_loop(0, NC8 // 2, chunk8, 0)
    wait_scatter(1)
    plsc.subcore_barrier()
    dump_acc(e8out_hbm)
    zero_acc()
    for r in range(RPT * H // 1280):
        pltpu.sync_copy(z1_v, den_sh.at[pl.ds(dbase + r * 1280, 1280)])
    plsc.subcore_barrier()

    issue_idx(gati_hbm, 0, 0)

    def al_copies(b):
        for h in range(H):
            yield (als_hbm.at[sgi_v.at[b, h]], als_g.at[b, h], al[b])
            yield (ald_hbm.at[dgi_v.at[b, h]], ald_g.at[b, h], al[b])

    def chunkg(jj, carry):
        for b in range(2):
            j = 2 * jj + b
            b1 = 1 - b
            wait_idx(gati_hbm, j, b)

            @pl.when(j >= 2)
            def _():
                for h in range(H):
                    pltpu.make_async_copy(exv_v.at[b, h],
                                          den_sh.at[dgi_v.at[b, h]],
                                          dn[b]).wait()

            for g in range(B // 16):
                sl = pl.ds(g * 16, 16)
                sv = gs_c[b, 0, sl]
                gio_v[b, sl] = sv + off
                dv = gs_c[b, 1, sl]
                sv4 = sv * H
                dv4 = dv * H
                for h in range(H):
                    sgi_v[b, h, sl] = sv4 + h
                    dgi_v[b, h, sl] = dv4 + h
            for args in al_copies(b):
                pltpu.async_copy(*args)
            pltpu.async_copy(xl_hbm.at[gio_v.at[b]], rows_v.at[b], fg[b])

            @pl.when(j >= 1)
            def _():
                wait_scatter(b1)

            @pl.when(j + 1 < NCG)
            def _():
                issue_idx(gati_hbm, j + 1, b1)

            for args in al_copies(b):
                pltpu.make_async_copy(*args).wait()
            for g in range(B // 16):
                sl = pl.ds(g * 16, 16)
                for h in range(H):
                    a = als_g[b, h, sl] + ald_g[b, h, sl]
                    a = jnp.where(a > 0.0, a, 0.2 * a)
                    exv_v[b, h, sl] = jnp.exp(a)
            for h in range(H):
                pltpu.async_copy(exv_v.at[b, h], den_sh.at[dgi_v.at[b, h]],
                                 dn[b], add=True)
            pltpu.make_async_copy(xl_hbm.at[gio_v.at[b]], rows_v.at[b],
                                  fg[b]).wait()
            for g in range(B // 16):
                sl = pl.ds(g * 16, 16)
                s0 = exv_v[b, c2, sl]
                s1 = exv_v[b, c2 + 1, sl]

                def edge(e, cc):
                    m0 = _take16(s0, jnp.full((16,), e, i32))
                    m1 = _take16(s1, jnp.full((16,), e, i32))
                    for k in range(4):
                        ksl = pl.ds(k * 16, 16)
                        rows_v[b, g * 16 + e, ksl] = \
                            rows_v[b, g * 16 + e, ksl] * m0
                    for k in range(4, 8):
                        ksl = pl.ds(k * 16, 16)
                        rows_v[b, g * 16 + e, ksl] = \
                            rows_v[b, g * 16 + e, ksl] * m1
                    return cc

                lax.fori_loop(0, 16, edge, 0)
            pltpu.async_copy(rows_v.at[b], acc_sh.at[gs_c.at[b, 1]], st[b],
                             add=True)
        return carry

    lax.fori_loop(0, NCG // 2, chunkg, 0)
    wait_scatter(1)
    for b in range(2):
        for h in range(H):
            pltpu.make_async_copy(exv_v.at[b, h], den_sh.at[dgi_v.at[b, h]],
                                  dn[b]).wait()
    plsc.subcore_barrier()
    dump_acc(gatout_hbm)

    @pl.when(c == 0)
    def _():
        pltpu.sync_copy(den_sh.at[pl.ds(dbase, RPT * H)],
                        den_hbm.at[pl.ds(dbase, RPT * H)])


def _row_to_col(row):
    m = jnp.broadcast_to(row, (BM, BM))
    ri = lax.broadcasted_iota(i32, (BM, BM), 0)
    ci = lax.broadcasted_iota(i32, (BM, BM), 1)
    return jnp.sum(jnp.where(ri == ci, m, 0.0), axis=1, keepdims=True)


def _dinv_col(deg_row):
    deg = _row_to_col(deg_row)
    return jnp.where(deg > 0.0, lax.rsqrt(jnp.maximum(deg, 1e-12)), 0.0)


def _prep_body(with_emb):
    def body(*refs):
        if with_emb:
            (x_ref, wembT_ref, bemb_ref, wg_ref, asrc_ref, adst_ref, deg_ref,
             h_ref, hecat_ref, xlcat_ref, als_ref, ald_ref) = refs
            h = jnp.dot(x_ref[...], wembT_ref[...],
                        preferred_element_type=f32) + bemb_ref[...]
            h_ref[...] = h
        else:
            (x_ref, wg_ref, asrc_ref, adst_ref, deg_ref,
             hecat_ref, xlcat_ref, als_ref, ald_ref) = refs
            h = x_ref[...]
        dinv = _dinv_col(deg_ref[0])
        he = h * dinv
        hecat_ref[0] = he[:, :128]
        hecat_ref[1] = he[:, 128:]
        xl = jnp.dot(h, wg_ref[...], preferred_element_type=f32)
        xlcat_ref[0] = xl[:, :128]
        xlcat_ref[1] = xl[:, 128:]
        als_ref[...] = jnp.concatenate(
            [(xl[:, 64 * k:64 * (k + 1)] * asrc_ref[:, 64 * k:64 * (k + 1)]
              ).sum(-1, keepdims=True) for k in range(4)], axis=1)
        ald_ref[...] = jnp.concatenate(
            [(xl[:, 64 * k:64 * (k + 1)] * adst_ref[:, 64 * k:64 * (k + 1)]
              ).sum(-1, keepdims=True) for k in range(4)], axis=1)
    return body


def _wspec(shape):
    return pl.BlockSpec(shape, lambda i: (0,) * len(shape))


_PREP_OUTS = [
    jax.ShapeDtypeStruct((2, N_PAD, 128), f32),
    jax.ShapeDtypeStruct((2, N_PAD, 128), f32),
    jax.ShapeDtypeStruct((N_PAD, H), f32),
    jax.ShapeDtypeStruct((N_PAD, H), f32),
]
_PREP_OUT_SPECS = [
    pl.BlockSpec((2, BM, 128), lambda i: (0, i, 0)),
    pl.BlockSpec((2, BM, 128), lambda i: (0, i, 0)),
    pl.BlockSpec((BM, H), lambda i: (i, 0)),
    pl.BlockSpec((BM, H), lambda i: (i, 0)),
]

_prep_emb = pl.pallas_call(
    _prep_body(True),
    grid=(NBLK,),
    in_specs=[
        pl.BlockSpec((BM, D), lambda i: (i, 0)),
        _wspec((D, D)),
        _wspec((1, D)),
        _wspec((D, D)),
        _wspec((1, D)),
        _wspec((1, D)),
        pl.BlockSpec((1, 1, BM), lambda i: (i, 0, 0)),
    ],
    out_specs=[pl.BlockSpec((BM, D), lambda i: (i, 0))] + _PREP_OUT_SPECS,
    out_shape=[jax.ShapeDtypeStruct((N_PAD, D), f32)] + _PREP_OUTS,
)

_prep_noemb = pl.pallas_call(
    _prep_body(False),
    grid=(NBLK,),
    in_specs=[
        pl.BlockSpec((BM, D), lambda i: (i, 0)),
        _wspec((D, D)),
        _wspec((1, D)),
        _wspec((1, D)),
        pl.BlockSpec((1, 1, BM), lambda i: (i, 0, 0)),
    ],
    out_specs=_PREP_OUT_SPECS,
    out_shape=_PREP_OUTS,
)


def _post_body(e8_ref, gat_ref, den_ref, deg_ref, weT_ref, wfT_ref, bf_ref,
               bg_ref, res_ref, g_ref, bn_ref, out_ref):
    dinv = _dinv_col(deg_ref[0])
    agg = jnp.concatenate([e8_ref[0], e8_ref[1]], axis=1) * dinv
    xe = jnp.dot(agg, weT_ref[...], preferred_element_type=f32)
    gato = jnp.concatenate([gat_ref[0], gat_ref[1]], axis=1)
    xg = jnp.concatenate(
        [gato[:, 64 * k:64 * (k + 1)] / (den_ref[:, k:k + 1] + 1e-16)
         for k in range(4)], axis=1) + bg_ref[...]
    wfT = wfT_ref[...]
    h2 = (jnp.dot(xe, wfT[:D], preferred_element_type=f32)
          + jnp.dot(xg, wfT[D:], preferred_element_type=f32)
          + bf_ref[...] + res_ref[...])
    m = jnp.mean(h2, axis=-1, keepdims=True)
    v = jnp.mean((h2 - m) ** 2, axis=-1, keepdims=True)
    out_ref[...] = jnp.maximum(
        (h2 - m) / jnp.sqrt(v + 1e-5) * g_ref[...] + bn_ref[...], 0.0)


_post = pl.pallas_call(
    _post_body,
    grid=(NBLK,),
    in_specs=[
        pl.BlockSpec((2, BM, 128), lambda i: (0, i, 0)),
        pl.BlockSpec((2, BM, 128), lambda i: (0, i, 0)),
        pl.BlockSpec((BM, H), lambda i: (i, 0)),
        pl.BlockSpec((1, 1, BM), lambda i: (i, 0, 0)),
        _wspec((D, D)),
        _wspec((2 * D, D)),
        _wspec((1, D)),
        _wspec((1, D)),
        pl.BlockSpec((BM, D), lambda i: (i, 0)),
        _wspec((1, D)),
        _wspec((1, D)),
    ],
    out_specs=pl.BlockSpec((BM, D), lambda i: (i, 0)),
    out_shape=jax.ShapeDtypeStruct((N_PAD, D), f32),
)


def _readout_body(h_ref, wr1T_ref, br1_ref, wr2_ref, br2_ref, out_ref):
    r = jnp.maximum(
        jnp.dot(h_ref[...], wr1T_ref[...], preferred_element_type=f32)
        + br1_ref[...], 0.0)
    o = (r * wr2_ref[...]).sum(-1, keepdims=True) + br2_ref[...]
    out_ref[...] = 1.0 / (1.0 + jnp.exp(-o))


_readout = pl.pallas_call(
    _readout_body,
    grid=(NBLK,),
    in_specs=[
        pl.BlockSpec((BM, D), lambda i: (i, 0)),
        _wspec((D, D // 2)),
        _wspec((1, D // 2)),
        _wspec((1, D // 2)),
        _wspec((1, 1)),
    ],
    out_specs=pl.BlockSpec((BM, 1), lambda i: (i, 0)),
    out_shape=jax.ShapeDtypeStruct((N_PAD, 1), f32),
)


def _pad_idx(idx, tot):
    return jnp.concatenate(
        [idx, jnp.full((tot - idx.shape[0],), DUMMY, idx.dtype)])


def kernel(x, edge_index, Wemb, bemb, We1, Wg1, asrc1, adst1, bg1, Wf1, bf1,
           g1, bn1, We2, Wg2, asrc2, adst2, bg2, Wf2, bf2, g2, bn2, Wr1, br1,
           Wr2, br2):
    ei = edge_index.astype(i32)
    row, col = ei[0], ei[1]
    loop = jnp.arange(N, dtype=i32)
    g8 = _pad_idx(col, E8P).reshape(NTILE, NC8, 1, B)
    s8 = _pad_idx(row, E8P).reshape(NTILE, NC8, 1, B)
    e8i = jnp.concatenate([g8, s8], axis=2)
    sg = _pad_idx(jnp.concatenate([row, loop]), EGP).reshape(NTILE, NCG, 1, B)
    dg = _pad_idx(jnp.concatenate([col, loop]), EGP).reshape(NTILE, NCG, 1, B)
    gati = jnp.concatenate([sg, dg], axis=2)

    x_p = jnp.pad(x, ((0, N_PAD - N), (0, 0)))

    r2 = lambda a: a.reshape(1, -1)

    deg = _deg_kernel(g8.reshape(NTILE, NC8, B)).reshape(NBLK, 1, BM)

    def layer(h_or_x, Wemb_args, Wg, asrc, adst, bg, We, Wf, bf, g, bn):
        if Wemb_args is not None:
            WembT, bemb_ = Wemb_args
            h, hecat, xlcat, als, ald = _prep_emb(
                h_or_x, WembT, r2(bemb_), Wg, r2(asrc), r2(adst), deg)
        else:
            h = h_or_x
            hecat, xlcat, als, ald = _prep_noemb(
                h, Wg, r2(asrc), r2(adst), deg)
        hecat_f = hecat.reshape(NSC * N_PAD, 128)
        xlcat_f = xlcat.reshape(NSC * N_PAD, 128)
        e8o, gato, den = _layer_sc(e8i, gati, hecat_f, xlcat_f,
                                   als.reshape(N_PAD * H),
                                   ald.reshape(N_PAD * H))
        return _post(e8o.reshape(NSC, N_PAD, 128),
                     gato.reshape(NSC, N_PAD, 128),
                     den.reshape(N_PAD, H), deg, We.T, Wf.T, r2(bf), r2(bg),
                     h, r2(g), r2(bn))

    h1 = layer(x_p, (Wemb.T, bemb), Wg1, asrc1.reshape(1, D),
               adst1.reshape(1, D), bg1, We1, Wf1, bf1, g1, bn1)
    h2 = layer(h1, None, Wg2, asrc2.reshape(1, D),
               adst2.reshape(1, D), bg2, We2, Wf2, bf2, g2, bn2)
    out = _readout(h2, Wr1.T, r2(br1), Wr2, br2.reshape(1, 1))
    return out[:N]


# speedup vs baseline: 29.8065x; 1.0534x over previous
import functools

import jax
import jax.numpy as jnp
from jax import lax
from jax.experimental import pallas as pl
from jax.experimental.pallas import tpu as pltpu
from jax.experimental.pallas import tpu_sc as plsc

N = 10000
D = 256
H = 4
C = 64
N_PAD = 10240
DUMMY = N
B = 128
NSC = 2
NTILE = 16
E8P = 163840
EGP = 172032
NC8 = E8P // (NTILE * B)
NCG = EGP // (NTILE * B)
RPT = N_PAD // NTILE
BM = 256
NBLK = N_PAD // BM

f32 = jnp.float32
i32 = jnp.int32


def _take16(vec, idx16):
    dnums = lax.GatherDimensionNumbers(
        offset_dims=(), collapsed_slice_dims=(0,), start_index_map=(0,))
    return lax.gather(vec, idx16[:, None], dnums, slice_sizes=(1,),
                      mode=lax.GatherScatterMode.PROMISE_IN_BOUNDS)


_MESH = plsc.VectorSubcoreMesh(
    core_axis_name="c", subcore_axis_name="s",
    num_cores=NSC, num_subcores=NTILE)

_SC_PARAMS = pltpu.CompilerParams(needs_layout_passes=False)

def _fill(ref, rows, width, scalar):
    value = jnp.full((16,), scalar, f32)
    for r in range(rows):
        for k in range(width // 16):
            if rows == 1:
                ref[pl.ds(k * 16, 16)] = value
            else:
                ref[r, pl.ds(k * 16, 16)] = value


@functools.partial(
    pl.kernel,
    out_type=jax.ShapeDtypeStruct((N_PAD,), f32),
    mesh=_MESH,
    compiler_params=_SC_PARAMS,
    scratch_types=[
        pltpu.VMEM((B,), i32),
        pltpu.VMEM((B,), f32),
        pltpu.VMEM((RPT,), f32),
        pltpu.VMEM_SHARED((N_PAD,), f32),
    ])
def _deg_kernel(gi_hbm, out_hbm, gi_c, ones_v, z_v, acc_sh):
    c = lax.axis_index("c")
    s = lax.axis_index("s")
    base = s * RPT
    _fill(ones_v, 1, B, 1.0)
    _fill(z_v, 1, RPT, 0.0)
    pltpu.sync_copy(z_v, acc_sh.at[pl.ds(base, RPT)])
    plsc.subcore_barrier()

    def chunk(j, carry):
        pltpu.sync_copy(gi_hbm.at[s, j], gi_c)
        pltpu.sync_copy(ones_v, acc_sh.at[gi_c], add=True)
        return carry

    lax.fori_loop(0, NC8, chunk, 0)
    plsc.subcore_barrier()

    @pl.when(c == 0)
    def _():
        pltpu.sync_copy(acc_sh.at[pl.ds(base, RPT)], out_hbm.at[pl.ds(base, RPT)])


@functools.partial(
    pl.kernel,
    out_type=[
        jax.ShapeDtypeStruct((NSC * N_PAD, 128), f32),
        jax.ShapeDtypeStruct((NSC * N_PAD, 128), f32),
        jax.ShapeDtypeStruct((N_PAD * H,), f32),
    ],
    mesh=_MESH,
    compiler_params=_SC_PARAMS,
    scratch_types=[
        pltpu.VMEM((2, 2, B), i32),
        pltpu.VMEM((2, B), i32),
        pltpu.VMEM((2, B, 128), f32),
        pltpu.VMEM((32, 128), f32),
        pltpu.VMEM((2, H, B), i32),
        pltpu.VMEM((2, H, B), i32),
        pltpu.VMEM((2, H, B), f32),
        pltpu.VMEM((2, H, B), f32),
        pltpu.VMEM((2, H, B), f32),
        pltpu.VMEM((1280,), f32),
        pltpu.VMEM_SHARED((N_PAD, 128), f32),
        pltpu.VMEM_SHARED((N_PAD * H,), f32),
        pltpu.SemaphoreType.DMA,
        pltpu.SemaphoreType.DMA,
        pltpu.SemaphoreType.DMA,
        pltpu.SemaphoreType.DMA,
        pltpu.SemaphoreType.DMA,
        pltpu.SemaphoreType.DMA,
        pltpu.SemaphoreType.DMA,
        pltpu.SemaphoreType.DMA,
        pltpu.SemaphoreType.DMA,
        pltpu.SemaphoreType.DMA,
    ])
def _layer_sc(e8i_hbm, gati_hbm, he_hbm, xl_hbm, als_hbm, ald_hbm,
              e8out_hbm, gatout_hbm, den_hbm,
              gs_c, gio_v, rows_v, z128_v, sgi_v, dgi_v, als_g, ald_g,
              exv_v, z1_v, acc_sh, den_sh,
              im0, im1, al0, al1, fg0, fg1, dn0, dn1, st0, st1):
    c = lax.axis_index("c")
    s = lax.axis_index("s")
    base = s * RPT
    dbase = s * (RPT * H)
    off = jnp.full((16,), c * N_PAD, i32)
    c2 = 2 * c
    im = (im0, im1)
    al = (al0, al1)
    fg = (fg0, fg1)
    dn = (dn0, dn1)
    st = (st0, st1)
    _fill(z128_v, 32, 128, 0.0)
    _fill(z1_v, 1, 1280, 0.0)

    def zero_acc():
        for r in range(RPT // 32):
            pltpu.async_copy(z128_v, acc_sh.at[pl.ds(base + r * 32, 32)],
                             st0 if r % 2 == 0 else st1)
        for r in range(RPT // 32):
            pltpu.make_async_copy(z128_v, acc_sh.at[pl.ds(base + r * 32, 32)],
                                  st0 if r % 2 == 0 else st1).wait()

    def dump_acc(out_hbm):
        pltpu.sync_copy(acc_sh.at[pl.ds(base, RPT)],
                        out_hbm.at[pl.ds(c * N_PAD + base, RPT)])

    def wait_idx(idx_hbm, j, b):
        pltpu.make_async_copy(idx_hbm.at[s, j], gs_c.at[b], im[b]).wait()

    def issue_idx(idx_hbm, j, b):
        pltpu.async_copy(idx_hbm.at[s, j], gs_c.at[b], im[b])

    def wait_scatter(b):
        pltpu.make_async_copy(rows_v.at[b], acc_sh.at[gs_c.at[b, 1]],
                              st[b]).wait()

    zero_acc()
    plsc.subcore_barrier()
    issue_idx(e8i_hbm, 0, 0)

    def chunk8(jj, carry):
        for b in range(2):
            j = 2 * jj + b
            b1 = 1 - b
            wait_idx(e8i_hbm, j, b)
            for g in range(B // 16):
                sl = pl.ds(g * 16, 16)
                gio_v[b, sl] = gs_c[b, 0, sl] + off
            pltpu.async_copy(he_hbm.at[gio_v.at[b]], rows_v.at[b], fg[b])

            @pl.when(j >= 1)
            def _():
                wait_scatter(b1)

            @pl.when(j + 1 < NC8)
            def _():
                issue_idx(e8i_hbm, j + 1, b1)

            pltpu.make_async_copy(he_hbm.at[gio_v.at[b]], rows_v.at[b],
                                  fg[b]).wait()
            pltpu.async_copy(rows_v.at[b], acc_sh.at[gs_c.at[b, 1]], st[b],
                             add=True)
        return carry

    lax.fori_loop(0, NC8 // 2, chunk8, 0)
    wait_scatter(1)
    plsc.subcore_barrier()
    dump_acc(e8out_hbm)
    zero_acc()
    for r in range(RPT * H // 1280):
        pltpu.sync_copy(z1_v, den_sh.at[pl.ds(dbase + r * 1280, 1280)])
    plsc.subcore_barrier()

    issue_idx(gati_hbm, 0, 0)

    def al_copies(b):
        for h in range(H):
            yield (als_hbm.at[sgi_v.at[b, h]], als_g.at[b, h], al[b])
            yield (ald_hbm.at[dgi_v.at[b, h]], ald_g.at[b, h], al[b])

    def chunkg(jj, carry):
        for b in range(2):
            j = 2 * jj + b
            b1 = 1 - b
            wait_idx(gati_hbm, j, b)

            @pl.when(j >= 2)
            def _():
                for h in range(H):
                    pltpu.make_async_copy(exv_v.at[b, h],
                                          den_sh.at[dgi_v.at[b, h]],
                                          dn[b]).wait()

            for g in range(B // 16):
                sl = pl.ds(g * 16, 16)
                sv = gs_c[b, 0, sl]
                gio_v[b, sl] = sv + off
                dv = gs_c[b, 1, sl]
                sv4 = sv * H
                dv4 = dv * H
                for h in range(H):
                    sgi_v[b, h, sl] = sv4 + h
                    dgi_v[b, h, sl] = dv4 + h
            for args in al_copies(b):
                pltpu.async_copy(*args)
            pltpu.async_copy(xl_hbm.at[gio_v.at[b]], rows_v.at[b], fg[b])

            @pl.when(j >= 1)
            def _():
                wait_scatter(b1)

            @pl.when(j + 1 < NCG)
            def _():
                issue_idx(gati_hbm, j + 1, b1)

            for args in al_copies(b):
                pltpu.make_async_copy(*args).wait()
            for g in range(B // 16):
                sl = pl.ds(g * 16, 16)
                for h in range(H):
                    a = als_g[b, h, sl] + ald_g[b, h, sl]
                    a = jnp.where(a > 0.0, a, 0.2 * a)
                    exv_v[b, h, sl] = jnp.exp(a)
            for h in range(H):
                pltpu.async_copy(exv_v.at[b, h], den_sh.at[dgi_v.at[b, h]],
                                 dn[b], add=True)
            pltpu.make_async_copy(xl_hbm.at[gio_v.at[b]], rows_v.at[b],
                                  fg[b]).wait()
            for g in range(B // 16):
                sl = pl.ds(g * 16, 16)
                s0 = exv_v[b, c2, sl]
                s1 = exv_v[b, c2 + 1, sl]

                def edge(e, cc):
                    m0 = _take16(s0, jnp.full((16,), e, i32))
                    m1 = _take16(s1, jnp.full((16,), e, i32))
                    for k in range(4):
                        ksl = pl.ds(k * 16, 16)
                        rows_v[b, g * 16 + e, ksl] = \
                            rows_v[b, g * 16 + e, ksl] * m0
                    for k in range(4, 8):
                        ksl = pl.ds(k * 16, 16)
                        rows_v[b, g * 16 + e, ksl] = \
                            rows_v[b, g * 16 + e, ksl] * m1
                    return cc

                lax.fori_loop(0, 16, edge, 0)
            pltpu.async_copy(rows_v.at[b], acc_sh.at[gs_c.at[b, 1]], st[b],
                             add=True)
        return carry

    lax.fori_loop(0, NCG // 2, chunkg, 0)
    wait_scatter(1)
    for b in range(2):
        for h in range(H):
            pltpu.make_async_copy(exv_v.at[b, h], den_sh.at[dgi_v.at[b, h]],
                                  dn[b]).wait()
    plsc.subcore_barrier()
    dump_acc(gatout_hbm)

    @pl.when(c == 0)
    def _():
        pltpu.sync_copy(den_sh.at[pl.ds(dbase, RPT * H)],
                        den_hbm.at[pl.ds(dbase, RPT * H)])


def _row_to_col(row):
    m = jnp.broadcast_to(row, (BM, BM))
    ri = lax.broadcasted_iota(i32, (BM, BM), 0)
    ci = lax.broadcasted_iota(i32, (BM, BM), 1)
    return jnp.sum(jnp.where(ri == ci, m, 0.0), axis=1, keepdims=True)


def _dinv_col(deg_row):
    deg = _row_to_col(deg_row)
    return jnp.where(deg > 0.0, lax.rsqrt(jnp.maximum(deg, 1e-12)), 0.0)


def _prep_block(h, deg_ref, wg_ref, asrc_ref, adst_ref,
                hecat_ref, xlcat_ref, als_ref, ald_ref):
    dinv = _dinv_col(deg_ref[0])
    he = h * dinv
    hecat_ref[0] = he[:, :128]
    hecat_ref[1] = he[:, 128:]
    xl = jnp.dot(h, wg_ref[...], preferred_element_type=f32)
    xlcat_ref[0] = xl[:, :128]
    xlcat_ref[1] = xl[:, 128:]
    als_ref[...] = jnp.concatenate(
        [(xl[:, 64 * k:64 * (k + 1)] * asrc_ref[:, 64 * k:64 * (k + 1)]
          ).sum(-1, keepdims=True) for k in range(4)], axis=1)
    ald_ref[...] = jnp.concatenate(
        [(xl[:, 64 * k:64 * (k + 1)] * adst_ref[:, 64 * k:64 * (k + 1)]
          ).sum(-1, keepdims=True) for k in range(4)], axis=1)


def _post_block(e8_ref, gat_ref, den_ref, deg_ref, weT_ref, wfT_ref, bf_ref,
                bg_ref, res_ref, g_ref, bn_ref):
    dinv = _dinv_col(deg_ref[0])
    agg = jnp.concatenate([e8_ref[0], e8_ref[1]], axis=1) * dinv
    xe = jnp.dot(agg, weT_ref[...], preferred_element_type=f32)
    gato = jnp.concatenate([gat_ref[0], gat_ref[1]], axis=1)
    xg = jnp.concatenate(
        [gato[:, 64 * k:64 * (k + 1)] / (den_ref[:, k:k + 1] + 1e-16)
         for k in range(4)], axis=1) + bg_ref[...]
    wfT = wfT_ref[...]
    h2 = (jnp.dot(xe, wfT[:D], preferred_element_type=f32)
          + jnp.dot(xg, wfT[D:], preferred_element_type=f32)
          + bf_ref[...] + res_ref[...])
    m = jnp.mean(h2, axis=-1, keepdims=True)
    v = jnp.mean((h2 - m) ** 2, axis=-1, keepdims=True)
    return jnp.maximum(
        (h2 - m) / jnp.sqrt(v + 1e-5) * g_ref[...] + bn_ref[...], 0.0)


def _wspec(shape):
    return pl.BlockSpec(shape, lambda i: (0,) * len(shape))


def _bspec(*shape):
    return pl.BlockSpec(shape, lambda i: (i,) + (0,) * (len(shape) - 1))


_PREP_OUTS = [
    jax.ShapeDtypeStruct((2, N_PAD, 128), f32),
    jax.ShapeDtypeStruct((2, N_PAD, 128), f32),
    jax.ShapeDtypeStruct((N_PAD, H), f32),
    jax.ShapeDtypeStruct((N_PAD, H), f32),
]
_PREP_OUT_SPECS = [
    pl.BlockSpec((2, BM, 128), lambda i: (0, i, 0)),
    pl.BlockSpec((2, BM, 128), lambda i: (0, i, 0)),
    _bspec(BM, H),
    _bspec(BM, H),
]
_POST_IN_SPECS = [
    pl.BlockSpec((2, BM, 128), lambda i: (0, i, 0)),
    pl.BlockSpec((2, BM, 128), lambda i: (0, i, 0)),
    _bspec(BM, H),
    pl.BlockSpec((1, 1, BM), lambda i: (i, 0, 0)),
    _wspec((D, D)),
    _wspec((2 * D, D)),
    _wspec((1, D)),
    _wspec((1, D)),
    _bspec(BM, D),
    _wspec((1, D)),
    _wspec((1, D)),
]


def _prep_emb_body(x_ref, wembT_ref, bemb_ref, wg_ref, asrc_ref, adst_ref,
                   deg_ref, h_ref, hecat_ref, xlcat_ref, als_ref, ald_ref):
    h = jnp.dot(x_ref[...], wembT_ref[...],
                preferred_element_type=f32) + bemb_ref[...]
    h_ref[...] = h
    _prep_block(h, deg_ref, wg_ref, asrc_ref, adst_ref,
                hecat_ref, xlcat_ref, als_ref, ald_ref)


_prep_emb = pl.pallas_call(
    _prep_emb_body,
    grid=(NBLK,),
    in_specs=[
        _bspec(BM, D),
        _wspec((D, D)),
        _wspec((1, D)),
        _wspec((D, D)),
        _wspec((1, D)),
        _wspec((1, D)),
        pl.BlockSpec((1, 1, BM), lambda i: (i, 0, 0)),
    ],
    out_specs=[_bspec(BM, D)] + _PREP_OUT_SPECS,
    out_shape=[jax.ShapeDtypeStruct((N_PAD, D), f32)] + _PREP_OUTS,
)


def _post_prep_body(*refs):
    (e8_ref, gat_ref, den_ref, deg_ref, weT_ref, wfT_ref, bf_ref, bg_ref,
     res_ref, g_ref, bn_ref, wg_ref, asrc_ref, adst_ref,
     h_ref, hecat_ref, xlcat_ref, als_ref, ald_ref) = refs
    h2 = _post_block(e8_ref, gat_ref, den_ref, deg_ref, weT_ref, wfT_ref,
                     bf_ref, bg_ref, res_ref, g_ref, bn_ref)
    h_ref[...] = h2
    _prep_block(h2, deg_ref, wg_ref, asrc_ref, adst_ref,
                hecat_ref, xlcat_ref, als_ref, ald_ref)


_post_prep = pl.pallas_call(
    _post_prep_body,
    grid=(NBLK,),
    in_specs=_POST_IN_SPECS + [_wspec((D, D)), _wspec((1, D)), _wspec((1, D))],
    out_specs=[_bspec(BM, D)] + _PREP_OUT_SPECS,
    out_shape=[jax.ShapeDtypeStruct((N_PAD, D), f32)] + _PREP_OUTS,
)


def _post_read_body(*refs):
    (e8_ref, gat_ref, den_ref, deg_ref, weT_ref, wfT_ref, bf_ref, bg_ref,
     res_ref, g_ref, bn_ref, wr1T_ref, br1_ref, wr2_ref, br2_ref,
     out_ref) = refs
    h2 = _post_block(e8_ref, gat_ref, den_ref, deg_ref, weT_ref, wfT_ref,
                     bf_ref, bg_ref, res_ref, g_ref, bn_ref)
    r = jnp.maximum(
        jnp.dot(h2, wr1T_ref[...], preferred_element_type=f32)
        + br1_ref[...], 0.0)
    o = (r * wr2_ref[...]).sum(-1, keepdims=True) + br2_ref[...]
    out_ref[...] = 1.0 / (1.0 + jnp.exp(-o))


_post_read = pl.pallas_call(
    _post_read_body,
    grid=(NBLK,),
    in_specs=_POST_IN_SPECS + [
        _wspec((D, D // 2)),
        _wspec((1, D // 2)),
        _wspec((1, D // 2)),
        _wspec((1, 1)),
    ],
    out_specs=_bspec(BM, 1),
    out_shape=jax.ShapeDtypeStruct((N_PAD, 1), f32),
)


def _pad_idx(idx, tot):
    return jnp.concatenate(
        [idx, jnp.full((tot - idx.shape[0],), DUMMY, idx.dtype)])


def kernel(x, edge_index, Wemb, bemb, We1, Wg1, asrc1, adst1, bg1, Wf1, bf1,
           g1, bn1, We2, Wg2, asrc2, adst2, bg2, Wf2, bf2, g2, bn2, Wr1, br1,
           Wr2, br2):
    ei = edge_index.astype(i32)
    row, col = ei[0], ei[1]
    loop = jnp.arange(N, dtype=i32)
    g8 = _pad_idx(col, E8P).reshape(NTILE, NC8, 1, B)
    s8 = _pad_idx(row, E8P).reshape(NTILE, NC8, 1, B)
    e8i = jnp.concatenate([g8, s8], axis=2)
    sg = _pad_idx(jnp.concatenate([row, loop]), EGP).reshape(NTILE, NCG, 1, B)
    dg = _pad_idx(jnp.concatenate([col, loop]), EGP).reshape(NTILE, NCG, 1, B)
    gati = jnp.concatenate([sg, dg], axis=2)

    x_p = jnp.pad(x, ((0, N_PAD - N), (0, 0)))

    r2 = lambda a: a.reshape(1, -1)

    deg = _deg_kernel(g8.reshape(NTILE, NC8, B)).reshape(NBLK, 1, BM)

    h1, hecat, xlcat, als, ald = _prep_emb(
        x_p, Wemb.T, r2(bemb), Wg1, asrc1.reshape(1, D),
        adst1.reshape(1, D), deg)
    e8o, gato, den = _layer_sc(
        e8i, gati, hecat.reshape(NSC * N_PAD, 128),
        xlcat.reshape(NSC * N_PAD, 128),
        als.reshape(N_PAD * H), ald.reshape(N_PAD * H))
    h2, hecat, xlcat, als, ald = _post_prep(
        e8o.reshape(NSC, N_PAD, 128), gato.reshape(NSC, N_PAD, 128),
        den.reshape(N_PAD, H), deg, We1.T, Wf1.T, r2(bf1), r2(bg1), h1,
        r2(g1), r2(bn1), Wg2, asrc2.reshape(1, D), adst2.reshape(1, D))
    e8o, gato, den = _layer_sc(
        e8i, gati, hecat.reshape(NSC * N_PAD, 128),
        xlcat.reshape(NSC * N_PAD, 128),
        als.reshape(N_PAD * H), ald.reshape(N_PAD * H))
    out = _post_read(
        e8o.reshape(NSC, N_PAD, 128), gato.reshape(NSC, N_PAD, 128),
        den.reshape(N_PAD, H), deg, We2.T, Wf2.T, r2(bf2), r2(bg2), h2,
        r2(g2), r2(bn2), Wr1.T, r2(br1), Wr2, br2.reshape(1, 1))
    return out[:N]


# speedup vs baseline: 29.8155x; 1.0003x over previous
import functools

import jax
import jax.numpy as jnp
from jax import lax
from jax.experimental import pallas as pl
from jax.experimental.pallas import tpu as pltpu
from jax.experimental.pallas import tpu_sc as plsc

N = 10000
D = 256
H = 4
C = 64
N_PAD = 10240
DUMMY = N
B = 128
NSC = 2
NTILE = 16
E8P = 163840
EGP = 172032
NC8 = E8P // (NTILE * B)
NCG = EGP // (NTILE * B)
RPT = N_PAD // NTILE
BM = 256
NBLK = N_PAD // BM

f32 = jnp.float32
i32 = jnp.int32


def _take16(vec, idx16):
    dnums = lax.GatherDimensionNumbers(
        offset_dims=(), collapsed_slice_dims=(0,), start_index_map=(0,))
    return lax.gather(vec, idx16[:, None], dnums, slice_sizes=(1,),
                      mode=lax.GatherScatterMode.PROMISE_IN_BOUNDS)


_MESH = plsc.VectorSubcoreMesh(
    core_axis_name="c", subcore_axis_name="s",
    num_cores=NSC, num_subcores=NTILE)

_SC_PARAMS = pltpu.CompilerParams(needs_layout_passes=False)

def _fill(ref, rows, width, scalar):
    value = jnp.full((16,), scalar, f32)
    for r in range(rows):
        for k in range(width // 16):
            if rows == 1:
                ref[pl.ds(k * 16, 16)] = value
            else:
                ref[r, pl.ds(k * 16, 16)] = value


@functools.partial(
    pl.kernel,
    out_type=jax.ShapeDtypeStruct((N_PAD,), f32),
    mesh=_MESH,
    compiler_params=_SC_PARAMS,
    scratch_types=[
        pltpu.VMEM((B,), i32),
        pltpu.VMEM((B,), f32),
        pltpu.VMEM((RPT,), f32),
        pltpu.VMEM_SHARED((N_PAD,), f32),
    ])
def _deg_kernel(gi_hbm, out_hbm, gi_c, ones_v, z_v, acc_sh):
    c = lax.axis_index("c")
    s = lax.axis_index("s")
    base = s * RPT
    _fill(ones_v, 1, B, 1.0)
    _fill(z_v, 1, RPT, 0.0)
    pltpu.sync_copy(z_v, acc_sh.at[pl.ds(base, RPT)])
    plsc.subcore_barrier()

    def chunk(j, carry):
        pltpu.sync_copy(gi_hbm.at[s, j], gi_c)
        pltpu.sync_copy(ones_v, acc_sh.at[gi_c], add=True)
        return carry

    lax.fori_loop(0, NC8, chunk, 0)
    plsc.subcore_barrier()

    @pl.when(c == 0)
    def _():
        pltpu.sync_copy(acc_sh.at[pl.ds(base, RPT)], out_hbm.at[pl.ds(base, RPT)])


@functools.partial(
    pl.kernel,
    out_type=[
        jax.ShapeDtypeStruct((NSC * N_PAD, 128), f32),
        jax.ShapeDtypeStruct((NSC * N_PAD, 128), f32),
        jax.ShapeDtypeStruct((N_PAD * H,), f32),
    ],
    mesh=_MESH,
    compiler_params=_SC_PARAMS,
    scratch_types=[
        pltpu.VMEM((2, 2, B), i32),
        pltpu.VMEM((2, B), i32),
        pltpu.VMEM((2, B), i32),
        pltpu.VMEM((2, B, 128), f32),
        pltpu.VMEM((32, 128), f32),
        pltpu.VMEM((2, H, B), i32),
        pltpu.VMEM((2, H, B), i32),
        pltpu.VMEM((2, H, B), f32),
        pltpu.VMEM((2, H, B), f32),
        pltpu.VMEM((2, H, B), f32),
        pltpu.VMEM((1280,), f32),
        pltpu.VMEM_SHARED((N_PAD, 128), f32),
        pltpu.VMEM_SHARED((N_PAD * H,), f32),
        pltpu.SemaphoreType.DMA,
        pltpu.SemaphoreType.DMA,
        pltpu.SemaphoreType.DMA,
        pltpu.SemaphoreType.DMA,
        pltpu.SemaphoreType.DMA,
        pltpu.SemaphoreType.DMA,
        pltpu.SemaphoreType.DMA,
        pltpu.SemaphoreType.DMA,
        pltpu.SemaphoreType.DMA,
        pltpu.SemaphoreType.DMA,
    ])
def _layer_sc(e8i_hbm, gati_hbm, he_hbm, xl_hbm, als_hbm, ald_hbm,
              e8out_hbm, gatout_hbm, den_hbm,
              gs_c, gio_v, sct_v, rows_v, z128_v, sgi_v, dgi_v, als_g, ald_g,
              exv_v, z1_v, acc_sh, den_sh,
              im0, im1, al0, al1, fg0, fg1, dn0, dn1, st0, st1):
    c = lax.axis_index("c")
    s = lax.axis_index("s")
    base = s * RPT
    dbase = s * (RPT * H)
    off = jnp.full((16,), c * N_PAD, i32)
    c2 = 2 * c
    im = (im0, im1)
    al = (al0, al1)
    fg = (fg0, fg1)
    dn = (dn0, dn1)
    st = (st0, st1)
    _fill(z128_v, 32, 128, 0.0)
    _fill(z1_v, 1, 1280, 0.0)

    def zero_acc():
        for r in range(RPT // 32):
            pltpu.async_copy(z128_v, acc_sh.at[pl.ds(base + r * 32, 32)],
                             st0 if r % 2 == 0 else st1)
        for r in range(RPT // 32):
            pltpu.make_async_copy(z128_v, acc_sh.at[pl.ds(base + r * 32, 32)],
                                  st0 if r % 2 == 0 else st1).wait()

    def dump_acc(out_hbm):
        pltpu.sync_copy(acc_sh.at[pl.ds(base, RPT)],
                        out_hbm.at[pl.ds(c * N_PAD + base, RPT)])

    def wait_idx(idx_hbm, j, b):
        pltpu.make_async_copy(idx_hbm.at[s, j], gs_c.at[b], im[b]).wait()

    def issue_idx(idx_hbm, j, b):
        pltpu.async_copy(idx_hbm.at[s, j], gs_c.at[b], im[b])

    def wait_scatter(b):
        pltpu.make_async_copy(rows_v.at[b], acc_sh.at[sct_v.at[b]],
                              st[b]).wait()

    def issue_scatter(b):
        pltpu.async_copy(rows_v.at[b], acc_sh.at[sct_v.at[b]], st[b],
                         add=True)

    zero_acc()
    plsc.subcore_barrier()
    issue_idx(e8i_hbm, 0, 0)

    def chunk8(jj, carry):
        for b in range(2):
            j = 2 * jj + b
            b1 = 1 - b
            wait_idx(e8i_hbm, j, b)

            @pl.when(j >= 2)
            def _():
                wait_scatter(b)

            for g in range(B // 16):
                sl = pl.ds(g * 16, 16)
                gio_v[b, sl] = gs_c[b, 0, sl] + off
                sct_v[b, sl] = gs_c[b, 1, sl]
            pltpu.async_copy(he_hbm.at[gio_v.at[b]], rows_v.at[b], fg[b])

            @pl.when(j + 1 < NC8)
            def _():
                issue_idx(e8i_hbm, j + 1, b1)

            pltpu.make_async_copy(he_hbm.at[gio_v.at[b]], rows_v.at[b],
                                  fg[b]).wait()
            issue_scatter(b)
        return carry

    lax.fori_loop(0, NC8 // 2, chunk8, 0)
    wait_scatter(0)
    wait_scatter(1)
    plsc.subcore_barrier()
    dump_acc(e8out_hbm)
    zero_acc()
    for r in range(RPT * H // 1280):
        pltpu.sync_copy(z1_v, den_sh.at[pl.ds(dbase + r * 1280, 1280)])
    plsc.subcore_barrier()

    issue_idx(gati_hbm, 0, 0)

    def al_copies(b):
        for h in range(H):
            yield (als_hbm.at[sgi_v.at[b, h]], als_g.at[b, h], al[b])
            yield (ald_hbm.at[dgi_v.at[b, h]], ald_g.at[b, h], al[b])

    def chunkg(jj, carry):
        for b in range(2):
            j = 2 * jj + b
            b1 = 1 - b
            wait_idx(gati_hbm, j, b)

            @pl.when(j >= 2)
            def _():
                for h in range(H):
                    pltpu.make_async_copy(exv_v.at[b, h],
                                          den_sh.at[dgi_v.at[b, h]],
                                          dn[b]).wait()
                wait_scatter(b)

            for g in range(B // 16):
                sl = pl.ds(g * 16, 16)
                sv = gs_c[b, 0, sl]
                gio_v[b, sl] = sv + off
                dv = gs_c[b, 1, sl]
                sv4 = sv * H
                dv4 = dv * H
                sct_v[b, sl] = dv
                for h in range(H):
                    sgi_v[b, h, sl] = sv4 + h
                    dgi_v[b, h, sl] = dv4 + h
            for args in al_copies(b):
                pltpu.async_copy(*args)
            pltpu.async_copy(xl_hbm.at[gio_v.at[b]], rows_v.at[b], fg[b])

            @pl.when(j + 1 < NCG)
            def _():
                issue_idx(gati_hbm, j + 1, b1)

            for args in al_copies(b):
                pltpu.make_async_copy(*args).wait()
            for g in range(B // 16):
                sl = pl.ds(g * 16, 16)
                for h in range(H):
                    a = als_g[b, h, sl] + ald_g[b, h, sl]
                    a = jnp.where(a > 0.0, a, 0.2 * a)
                    exv_v[b, h, sl] = jnp.exp(a)
            for h in range(H):
                pltpu.async_copy(exv_v.at[b, h], den_sh.at[dgi_v.at[b, h]],
                                 dn[b], add=True)
            pltpu.make_async_copy(xl_hbm.at[gio_v.at[b]], rows_v.at[b],
                                  fg[b]).wait()
            for g in range(B // 16):
                sl = pl.ds(g * 16, 16)
                s0 = exv_v[b, c2, sl]
                s1 = exv_v[b, c2 + 1, sl]

                def edge(e, cc):
                    m0 = _take16(s0, jnp.full((16,), e, i32))
                    m1 = _take16(s1, jnp.full((16,), e, i32))
                    for k in range(4):
                        ksl = pl.ds(k * 16, 16)
                        rows_v[b, g * 16 + e, ksl] = \
                            rows_v[b, g * 16 + e, ksl] * m0
                    for k in range(4, 8):
                        ksl = pl.ds(k * 16, 16)
                        rows_v[b, g * 16 + e, ksl] = \
                            rows_v[b, g * 16 + e, ksl] * m1
                    return cc

                lax.fori_loop(0, 16, edge, 0)
            issue_scatter(b)
        return carry

    lax.fori_loop(0, NCG // 2, chunkg, 0)
    wait_scatter(0)
    wait_scatter(1)
    for b in range(2):
        for h in range(H):
            pltpu.make_async_copy(exv_v.at[b, h], den_sh.at[dgi_v.at[b, h]],
                                  dn[b]).wait()
    plsc.subcore_barrier()
    dump_acc(gatout_hbm)

    @pl.when(c == 0)
    def _():
        pltpu.sync_copy(den_sh.at[pl.ds(dbase, RPT * H)],
                        den_hbm.at[pl.ds(dbase, RPT * H)])


def _row_to_col(row):
    m = jnp.broadcast_to(row, (BM, BM))
    ri = lax.broadcasted_iota(i32, (BM, BM), 0)
    ci = lax.broadcasted_iota(i32, (BM, BM), 1)
    return jnp.sum(jnp.where(ri == ci, m, 0.0), axis=1, keepdims=True)


def _dinv_col(deg_row):
    deg = _row_to_col(deg_row)
    return jnp.where(deg > 0.0, lax.rsqrt(jnp.maximum(deg, 1e-12)), 0.0)


def _prep_block(h, deg_ref, wg_ref, asrc_ref, adst_ref,
                hecat_ref, xlcat_ref, als_ref, ald_ref):
    dinv = _dinv_col(deg_ref[0])
    he = h * dinv
    hecat_ref[0] = he[:, :128]
    hecat_ref[1] = he[:, 128:]
    xl = jnp.dot(h, wg_ref[...], preferred_element_type=f32)
    xlcat_ref[0] = xl[:, :128]
    xlcat_ref[1] = xl[:, 128:]
    als_ref[...] = jnp.concatenate(
        [(xl[:, 64 * k:64 * (k + 1)] * asrc_ref[:, 64 * k:64 * (k + 1)]
          ).sum(-1, keepdims=True) for k in range(4)], axis=1)
    ald_ref[...] = jnp.concatenate(
        [(xl[:, 64 * k:64 * (k + 1)] * adst_ref[:, 64 * k:64 * (k + 1)]
          ).sum(-1, keepdims=True) for k in range(4)], axis=1)


def _post_block(e8_ref, gat_ref, den_ref, deg_ref, weT_ref, wfT_ref, bf_ref,
                bg_ref, res_ref, g_ref, bn_ref):
    dinv = _dinv_col(deg_ref[0])
    agg = jnp.concatenate([e8_ref[0], e8_ref[1]], axis=1) * dinv
    xe = jnp.dot(agg, weT_ref[...], preferred_element_type=f32)
    gato = jnp.concatenate([gat_ref[0], gat_ref[1]], axis=1)
    xg = jnp.concatenate(
        [gato[:, 64 * k:64 * (k + 1)] / (den_ref[:, k:k + 1] + 1e-16)
         for k in range(4)], axis=1) + bg_ref[...]
    wfT = wfT_ref[...]
    h2 = (jnp.dot(xe, wfT[:D], preferred_element_type=f32)
          + jnp.dot(xg, wfT[D:], preferred_element_type=f32)
          + bf_ref[...] + res_ref[...])
    m = jnp.mean(h2, axis=-1, keepdims=True)
    v = jnp.mean((h2 - m) ** 2, axis=-1, keepdims=True)
    return jnp.maximum(
        (h2 - m) / jnp.sqrt(v + 1e-5) * g_ref[...] + bn_ref[...], 0.0)


def _wspec(shape):
    return pl.BlockSpec(shape, lambda i: (0,) * len(shape))


def _bspec(*shape):
    return pl.BlockSpec(shape, lambda i: (i,) + (0,) * (len(shape) - 1))


_PREP_OUTS = [
    jax.ShapeDtypeStruct((2, N_PAD, 128), f32),
    jax.ShapeDtypeStruct((2, N_PAD, 128), f32),
    jax.ShapeDtypeStruct((N_PAD, H), f32),
    jax.ShapeDtypeStruct((N_PAD, H), f32),
]
_PREP_OUT_SPECS = [
    pl.BlockSpec((2, BM, 128), lambda i: (0, i, 0)),
    pl.BlockSpec((2, BM, 128), lambda i: (0, i, 0)),
    _bspec(BM, H),
    _bspec(BM, H),
]
_POST_IN_SPECS = [
    pl.BlockSpec((2, BM, 128), lambda i: (0, i, 0)),
    pl.BlockSpec((2, BM, 128), lambda i: (0, i, 0)),
    _bspec(BM, H),
    pl.BlockSpec((1, 1, BM), lambda i: (i, 0, 0)),
    _wspec((D, D)),
    _wspec((2 * D, D)),
    _wspec((1, D)),
    _wspec((1, D)),
    _bspec(BM, D),
    _wspec((1, D)),
    _wspec((1, D)),
]


def _prep_emb_body(x_ref, wembT_ref, bemb_ref, wg_ref, asrc_ref, adst_ref,
                   deg_ref, h_ref, hecat_ref, xlcat_ref, als_ref, ald_ref):
    h = jnp.dot(x_ref[...], wembT_ref[...],
                preferred_element_type=f32) + bemb_ref[...]
    h_ref[...] = h
    _prep_block(h, deg_ref, wg_ref, asrc_ref, adst_ref,
                hecat_ref, xlcat_ref, als_ref, ald_ref)


_prep_emb = pl.pallas_call(
    _prep_emb_body,
    grid=(NBLK,),
    in_specs=[
        _bspec(BM, D),
        _wspec((D, D)),
        _wspec((1, D)),
        _wspec((D, D)),
        _wspec((1, D)),
        _wspec((1, D)),
        pl.BlockSpec((1, 1, BM), lambda i: (i, 0, 0)),
    ],
    out_specs=[_bspec(BM, D)] + _PREP_OUT_SPECS,
    out_shape=[jax.ShapeDtypeStruct((N_PAD, D), f32)] + _PREP_OUTS,
)


def _post_prep_body(*refs):
    (e8_ref, gat_ref, den_ref, deg_ref, weT_ref, wfT_ref, bf_ref, bg_ref,
     res_ref, g_ref, bn_ref, wg_ref, asrc_ref, adst_ref,
     h_ref, hecat_ref, xlcat_ref, als_ref, ald_ref) = refs
    h2 = _post_block(e8_ref, gat_ref, den_ref, deg_ref, weT_ref, wfT_ref,
                     bf_ref, bg_ref, res_ref, g_ref, bn_ref)
    h_ref[...] = h2
    _prep_block(h2, deg_ref, wg_ref, asrc_ref, adst_ref,
                hecat_ref, xlcat_ref, als_ref, ald_ref)


_post_prep = pl.pallas_call(
    _post_prep_body,
    grid=(NBLK,),
    in_specs=_POST_IN_SPECS + [_wspec((D, D)), _wspec((1, D)), _wspec((1, D))],
    out_specs=[_bspec(BM, D)] + _PREP_OUT_SPECS,
    out_shape=[jax.ShapeDtypeStruct((N_PAD, D), f32)] + _PREP_OUTS,
)


def _post_read_body(*refs):
    (e8_ref, gat_ref, den_ref, deg_ref, weT_ref, wfT_ref, bf_ref, bg_ref,
     res_ref, g_ref, bn_ref, wr1T_ref, br1_ref, wr2_ref, br2_ref,
     out_ref) = refs
    h2 = _post_block(e8_ref, gat_ref, den_ref, deg_ref, weT_ref, wfT_ref,
                     bf_ref, bg_ref, res_ref, g_ref, bn_ref)
    r = jnp.maximum(
        jnp.dot(h2, wr1T_ref[...], preferred_element_type=f32)
        + br1_ref[...], 0.0)
    o = (r * wr2_ref[...]).sum(-1, keepdims=True) + br2_ref[...]
    out_ref[...] = 1.0 / (1.0 + jnp.exp(-o))


_post_read = pl.pallas_call(
    _post_read_body,
    grid=(NBLK,),
    in_specs=_POST_IN_SPECS + [
        _wspec((D, D // 2)),
        _wspec((1, D // 2)),
        _wspec((1, D // 2)),
        _wspec((1, 1)),
    ],
    out_specs=_bspec(BM, 1),
    out_shape=jax.ShapeDtypeStruct((N_PAD, 1), f32),
)


def _pad_idx(idx, tot):
    return jnp.concatenate(
        [idx, jnp.full((tot - idx.shape[0],), DUMMY, idx.dtype)])


def kernel(x, edge_index, Wemb, bemb, We1, Wg1, asrc1, adst1, bg1, Wf1, bf1,
           g1, bn1, We2, Wg2, asrc2, adst2, bg2, Wf2, bf2, g2, bn2, Wr1, br1,
           Wr2, br2):
    ei = edge_index.astype(i32)
    row, col = ei[0], ei[1]
    loop = jnp.arange(N, dtype=i32)
    g8 = _pad_idx(col, E8P).reshape(NTILE, NC8, 1, B)
    s8 = _pad_idx(row, E8P).reshape(NTILE, NC8, 1, B)
    e8i = jnp.concatenate([g8, s8], axis=2)
    sg = _pad_idx(jnp.concatenate([row, loop]), EGP).reshape(NTILE, NCG, 1, B)
    dg = _pad_idx(jnp.concatenate([col, loop]), EGP).reshape(NTILE, NCG, 1, B)
    gati = jnp.concatenate([sg, dg], axis=2)

    x_p = jnp.pad(x, ((0, N_PAD - N), (0, 0)))

    r2 = lambda a: a.reshape(1, -1)

    deg = _deg_kernel(g8.reshape(NTILE, NC8, B)).reshape(NBLK, 1, BM)

    h1, hecat, xlcat, als, ald = _prep_emb(
        x_p, Wemb.T, r2(bemb), Wg1, asrc1.reshape(1, D),
        adst1.reshape(1, D), deg)
    e8o, gato, den = _layer_sc(
        e8i, gati, hecat.reshape(NSC * N_PAD, 128),
        xlcat.reshape(NSC * N_PAD, 128),
        als.reshape(N_PAD * H), ald.reshape(N_PAD * H))
    h2, hecat, xlcat, als, ald = _post_prep(
        e8o.reshape(NSC, N_PAD, 128), gato.reshape(NSC, N_PAD, 128),
        den.reshape(N_PAD, H), deg, We1.T, Wf1.T, r2(bf1), r2(bg1), h1,
        r2(g1), r2(bn1), Wg2, asrc2.reshape(1, D), adst2.reshape(1, D))
    e8o, gato, den = _layer_sc(
        e8i, gati, hecat.reshape(NSC * N_PAD, 128),
        xlcat.reshape(NSC * N_PAD, 128),
        als.reshape(N_PAD * H), ald.reshape(N_PAD * H))
    out = _post_read(
        e8o.reshape(NSC, N_PAD, 128), gato.reshape(NSC, N_PAD, 128),
        den.reshape(N_PAD, H), deg, We2.T, Wf2.T, r2(bf2), r2(bg2), h2,
        r2(g2), r2(bn2), Wr1.T, r2(br1), Wr2, br2.reshape(1, 1))
    return out[:N]


# speedup vs baseline: 30.6643x; 1.0285x over previous
import functools

import jax
import jax.numpy as jnp
from jax import lax
from jax.experimental import pallas as pl
from jax.experimental.pallas import tpu as pltpu
from jax.experimental.pallas import tpu_sc as plsc

N = 10000
D = 256
H = 4
C = 64
N_PAD = 10240
DUMMY = N
B = 128
NSC = 2
NTILE = 16
E8P = 163840
EGP = 172032
NC8 = E8P // (NTILE * B)
NCG = EGP // (NTILE * B)
RPT = N_PAD // NTILE
BM = 256
NBLK = N_PAD // BM

f32 = jnp.float32
i32 = jnp.int32


def _take16(vec, idx16):
    dnums = lax.GatherDimensionNumbers(
        offset_dims=(), collapsed_slice_dims=(0,), start_index_map=(0,))
    return lax.gather(vec, idx16[:, None], dnums, slice_sizes=(1,),
                      mode=lax.GatherScatterMode.PROMISE_IN_BOUNDS)


_MESH = plsc.VectorSubcoreMesh(
    core_axis_name="c", subcore_axis_name="s",
    num_cores=NSC, num_subcores=NTILE)

_SC_PARAMS = pltpu.CompilerParams(needs_layout_passes=False)

def _fill(ref, rows, width, scalar):
    value = jnp.full((16,), scalar, f32)
    for r in range(rows):
        for k in range(width // 16):
            if rows == 1:
                ref[pl.ds(k * 16, 16)] = value
            else:
                ref[r, pl.ds(k * 16, 16)] = value


@functools.partial(
    pl.kernel,
    out_type=jax.ShapeDtypeStruct((N_PAD,), f32),
    mesh=_MESH,
    compiler_params=_SC_PARAMS,
    scratch_types=[
        pltpu.VMEM((B,), i32),
        pltpu.VMEM((B,), f32),
        pltpu.VMEM((RPT,), f32),
        pltpu.VMEM_SHARED((N_PAD,), f32),
    ])
def _deg_kernel(gi_hbm, out_hbm, gi_c, ones_v, z_v, acc_sh):
    c = lax.axis_index("c")
    s = lax.axis_index("s")
    base = s * RPT
    _fill(ones_v, 1, B, 1.0)
    _fill(z_v, 1, RPT, 0.0)
    pltpu.sync_copy(z_v, acc_sh.at[pl.ds(base, RPT)])
    plsc.subcore_barrier()

    def chunk(j, carry):
        pltpu.sync_copy(gi_hbm.at[s, j], gi_c)
        pltpu.sync_copy(ones_v, acc_sh.at[gi_c], add=True)
        return carry

    lax.fori_loop(0, NC8, chunk, 0)
    plsc.subcore_barrier()

    @pl.when(c == 0)
    def _():
        pltpu.sync_copy(acc_sh.at[pl.ds(base, RPT)], out_hbm.at[pl.ds(base, RPT)])


@functools.partial(
    pl.kernel,
    out_type=[
        jax.ShapeDtypeStruct((NSC * N_PAD, 128), f32),
        jax.ShapeDtypeStruct((NSC * N_PAD, 128), f32),
        jax.ShapeDtypeStruct((N_PAD * H,), f32),
    ],
    mesh=_MESH,
    compiler_params=_SC_PARAMS,
    scratch_types=[
        pltpu.VMEM((2, 2, B), i32),
        pltpu.VMEM((2, B), i32),
        pltpu.VMEM((2, B), i32),
        pltpu.VMEM((2, B, 128), f32),
        pltpu.VMEM((32, 128), f32),
        pltpu.VMEM((2, H, B), i32),
        pltpu.VMEM((2, H, B), i32),
        pltpu.VMEM((2, H, B), f32),
        pltpu.VMEM((2, H, B), f32),
        pltpu.VMEM((2, H, B), f32),
        pltpu.VMEM((1280,), f32),
        pltpu.VMEM_SHARED((N_PAD, 128), f32),
        pltpu.VMEM_SHARED((N_PAD * H,), f32),
        pltpu.SemaphoreType.DMA,
        pltpu.SemaphoreType.DMA,
        pltpu.SemaphoreType.DMA,
        pltpu.SemaphoreType.DMA,
        pltpu.SemaphoreType.DMA,
        pltpu.SemaphoreType.DMA,
        pltpu.SemaphoreType.DMA,
        pltpu.SemaphoreType.DMA,
        pltpu.SemaphoreType.DMA,
        pltpu.SemaphoreType.DMA,
    ])
def _layer_sc(e8i_hbm, gati_hbm, he_hbm, xl_hbm, als_hbm, ald_hbm,
              e8out_hbm, gatout_hbm, den_hbm,
              gs_c, gio_v, sct_v, rows_v, z128_v, sgi_v, dgi_v, als_g, ald_g,
              exv_v, z1_v, acc_sh, den_sh,
              im0, im1, al0, al1, fg0, fg1, dn0, dn1, st0, st1):
    c = lax.axis_index("c")
    s = lax.axis_index("s")
    base = s * RPT
    dbase = s * (RPT * H)
    off = jnp.full((16,), c * N_PAD, i32)
    c2 = 2 * c
    im = (im0, im1)
    al = (al0, al1)
    fg = (fg0, fg1)
    dn = (dn0, dn1)
    st = (st0, st1)
    _fill(z128_v, 32, 128, 0.0)
    _fill(z1_v, 1, 1280, 0.0)

    def zero_acc():
        for r in range(RPT // 32):
            pltpu.async_copy(z128_v, acc_sh.at[pl.ds(base + r * 32, 32)],
                             st0 if r % 2 == 0 else st1)
        for r in range(RPT // 32):
            pltpu.make_async_copy(z128_v, acc_sh.at[pl.ds(base + r * 32, 32)],
                                  st0 if r % 2 == 0 else st1).wait()

    def dump_acc(out_hbm):
        pltpu.sync_copy(acc_sh.at[pl.ds(base, RPT)],
                        out_hbm.at[pl.ds(c * N_PAD + base, RPT)])

    def wait_idx(idx_hbm, j, b):
        pltpu.make_async_copy(idx_hbm.at[s, j], gs_c.at[b], im[b]).wait()

    def issue_idx(idx_hbm, j, b):
        pltpu.async_copy(idx_hbm.at[s, j], gs_c.at[b], im[b])

    def wait_scatter(b):
        pltpu.make_async_copy(rows_v.at[b], acc_sh.at[sct_v.at[b]],
                              st[b]).wait()

    def issue_scatter(b):
        pltpu.async_copy(rows_v.at[b], acc_sh.at[sct_v.at[b]], st[b],
                         add=True)

    zero_acc()
    plsc.subcore_barrier()
    issue_idx(e8i_hbm, 0, 0)

    def chunk8(jj, carry):
        for b in range(2):
            j = 2 * jj + b
            b1 = 1 - b
            wait_idx(e8i_hbm, j, b)

            @pl.when(j >= 2)
            def _():
                wait_scatter(b)

            for g in range(B // 16):
                sl = pl.ds(g * 16, 16)
                gio_v[b, sl] = gs_c[b, 0, sl] + off
                sct_v[b, sl] = gs_c[b, 1, sl]
            pltpu.async_copy(he_hbm.at[gio_v.at[b]], rows_v.at[b], fg[b])

            @pl.when(j + 1 < NC8)
            def _():
                issue_idx(e8i_hbm, j + 1, b1)

            @pl.when(j >= 1)
            def _():
                pltpu.make_async_copy(he_hbm.at[gio_v.at[b1]], rows_v.at[b1],
                                      fg[b1]).wait()
                issue_scatter(b1)
        return carry

    lax.fori_loop(0, NC8 // 2, chunk8, 0)
    pltpu.make_async_copy(he_hbm.at[gio_v.at[1]], rows_v.at[1],
                          fg[1]).wait()
    issue_scatter(1)
    wait_scatter(0)
    wait_scatter(1)
    plsc.subcore_barrier()
    dump_acc(e8out_hbm)
    zero_acc()
    for r in range(RPT * H // 1280):
        pltpu.sync_copy(z1_v, den_sh.at[pl.ds(dbase + r * 1280, 1280)])
    plsc.subcore_barrier()

    issue_idx(gati_hbm, 0, 0)

    def al_copies(b):
        for h in range(H):
            yield (als_hbm.at[sgi_v.at[b, h]], als_g.at[b, h], al[b])
            yield (ald_hbm.at[dgi_v.at[b, h]], ald_g.at[b, h], al[b])

    def chunkg(jj, carry):
        for b in range(2):
            j = 2 * jj + b
            b1 = 1 - b
            wait_idx(gati_hbm, j, b)

            @pl.when(j >= 2)
            def _():
                for h in range(H):
                    pltpu.make_async_copy(exv_v.at[b, h],
                                          den_sh.at[dgi_v.at[b, h]],
                                          dn[b]).wait()
                wait_scatter(b)

            for g in range(B // 16):
                sl = pl.ds(g * 16, 16)
                sv = gs_c[b, 0, sl]
                gio_v[b, sl] = sv + off
                dv = gs_c[b, 1, sl]
                sv4 = sv * H
                dv4 = dv * H
                sct_v[b, sl] = dv
                for h in range(H):
                    sgi_v[b, h, sl] = sv4 + h
                    dgi_v[b, h, sl] = dv4 + h
            for args in al_copies(b):
                pltpu.async_copy(*args)
            pltpu.async_copy(xl_hbm.at[gio_v.at[b]], rows_v.at[b], fg[b])

            @pl.when(j + 1 < NCG)
            def _():
                issue_idx(gati_hbm, j + 1, b1)

            for args in al_copies(b):
                pltpu.make_async_copy(*args).wait()
            for g in range(B // 16):
                sl = pl.ds(g * 16, 16)
                for h in range(H):
                    a = als_g[b, h, sl] + ald_g[b, h, sl]
                    a = jnp.where(a > 0.0, a, 0.2 * a)
                    exv_v[b, h, sl] = jnp.exp(a)
            for h in range(H):
                pltpu.async_copy(exv_v.at[b, h], den_sh.at[dgi_v.at[b, h]],
                                 dn[b], add=True)
            pltpu.make_async_copy(xl_hbm.at[gio_v.at[b]], rows_v.at[b],
                                  fg[b]).wait()
            for g in range(B // 16):
                sl = pl.ds(g * 16, 16)
                s0 = exv_v[b, c2, sl]
                s1 = exv_v[b, c2 + 1, sl]

                def edge(e, cc):
                    m0 = _take16(s0, jnp.full((16,), e, i32))
                    m1 = _take16(s1, jnp.full((16,), e, i32))
                    for k in range(4):
                        ksl = pl.ds(k * 16, 16)
                        rows_v[b, g * 16 + e, ksl] = \
                            rows_v[b, g * 16 + e, ksl] * m0
                    for k in range(4, 8):
                        ksl = pl.ds(k * 16, 16)
                        rows_v[b, g * 16 + e, ksl] = \
                            rows_v[b, g * 16 + e, ksl] * m1
                    return cc

                lax.fori_loop(0, 16, edge, 0)
            issue_scatter(b)
        return carry

    lax.fori_loop(0, NCG // 2, chunkg, 0)
    wait_scatter(0)
    wait_scatter(1)
    for b in range(2):
        for h in range(H):
            pltpu.make_async_copy(exv_v.at[b, h], den_sh.at[dgi_v.at[b, h]],
                                  dn[b]).wait()
    plsc.subcore_barrier()
    dump_acc(gatout_hbm)

    @pl.when(c == 0)
    def _():
        pltpu.sync_copy(den_sh.at[pl.ds(dbase, RPT * H)],
                        den_hbm.at[pl.ds(dbase, RPT * H)])


def _row_to_col(row):
    m = jnp.broadcast_to(row, (BM, BM))
    ri = lax.broadcasted_iota(i32, (BM, BM), 0)
    ci = lax.broadcasted_iota(i32, (BM, BM), 1)
    return jnp.sum(jnp.where(ri == ci, m, 0.0), axis=1, keepdims=True)


def _dinv_col(deg_row):
    deg = _row_to_col(deg_row)
    return jnp.where(deg > 0.0, lax.rsqrt(jnp.maximum(deg, 1e-12)), 0.0)


def _prep_block(h, deg_ref, wg_ref, asrc_ref, adst_ref,
                hecat_ref, xlcat_ref, als_ref, ald_ref):
    dinv = _dinv_col(deg_ref[0])
    he = h * dinv
    hecat_ref[0] = he[:, :128]
    hecat_ref[1] = he[:, 128:]
    xl = jnp.dot(h, wg_ref[...], preferred_element_type=f32)
    xlcat_ref[0] = xl[:, :128]
    xlcat_ref[1] = xl[:, 128:]
    als_ref[...] = jnp.concatenate(
        [(xl[:, 64 * k:64 * (k + 1)] * asrc_ref[:, 64 * k:64 * (k + 1)]
          ).sum(-1, keepdims=True) for k in range(4)], axis=1)
    ald_ref[...] = jnp.concatenate(
        [(xl[:, 64 * k:64 * (k + 1)] * adst_ref[:, 64 * k:64 * (k + 1)]
          ).sum(-1, keepdims=True) for k in range(4)], axis=1)


def _post_block(e8_ref, gat_ref, den_ref, deg_ref, weT_ref, wfT_ref, bf_ref,
                bg_ref, res_ref, g_ref, bn_ref):
    dinv = _dinv_col(deg_ref[0])
    agg = jnp.concatenate([e8_ref[0], e8_ref[1]], axis=1) * dinv
    xe = jnp.dot(agg, weT_ref[...], preferred_element_type=f32)
    gato = jnp.concatenate([gat_ref[0], gat_ref[1]], axis=1)
    xg = jnp.concatenate(
        [gato[:, 64 * k:64 * (k + 1)] / (den_ref[:, k:k + 1] + 1e-16)
         for k in range(4)], axis=1) + bg_ref[...]
    wfT = wfT_ref[...]
    h2 = (jnp.dot(xe, wfT[:D], preferred_element_type=f32)
          + jnp.dot(xg, wfT[D:], preferred_element_type=f32)
          + bf_ref[...] + res_ref[...])
    m = jnp.mean(h2, axis=-1, keepdims=True)
    v = jnp.mean((h2 - m) ** 2, axis=-1, keepdims=True)
    return jnp.maximum(
        (h2 - m) / jnp.sqrt(v + 1e-5) * g_ref[...] + bn_ref[...], 0.0)


def _wspec(shape):
    return pl.BlockSpec(shape, lambda i: (0,) * len(shape))


def _bspec(*shape):
    return pl.BlockSpec(shape, lambda i: (i,) + (0,) * (len(shape) - 1))


_PREP_OUTS = [
    jax.ShapeDtypeStruct((2, N_PAD, 128), f32),
    jax.ShapeDtypeStruct((2, N_PAD, 128), f32),
    jax.ShapeDtypeStruct((N_PAD, H), f32),
    jax.ShapeDtypeStruct((N_PAD, H), f32),
]
_PREP_OUT_SPECS = [
    pl.BlockSpec((2, BM, 128), lambda i: (0, i, 0)),
    pl.BlockSpec((2, BM, 128), lambda i: (0, i, 0)),
    _bspec(BM, H),
    _bspec(BM, H),
]
_POST_IN_SPECS = [
    pl.BlockSpec((2, BM, 128), lambda i: (0, i, 0)),
    pl.BlockSpec((2, BM, 128), lambda i: (0, i, 0)),
    _bspec(BM, H),
    pl.BlockSpec((1, 1, BM), lambda i: (i, 0, 0)),
    _wspec((D, D)),
    _wspec((2 * D, D)),
    _wspec((1, D)),
    _wspec((1, D)),
    _bspec(BM, D),
    _wspec((1, D)),
    _wspec((1, D)),
]


def _prep_emb_body(x_ref, wembT_ref, bemb_ref, wg_ref, asrc_ref, adst_ref,
                   deg_ref, h_ref, hecat_ref, xlcat_ref, als_ref, ald_ref):
    h = jnp.dot(x_ref[...], wembT_ref[...],
                preferred_element_type=f32) + bemb_ref[...]
    h_ref[...] = h
    _prep_block(h, deg_ref, wg_ref, asrc_ref, adst_ref,
                hecat_ref, xlcat_ref, als_ref, ald_ref)


_prep_emb = pl.pallas_call(
    _prep_emb_body,
    grid=(NBLK,),
    in_specs=[
        _bspec(BM, D),
        _wspec((D, D)),
        _wspec((1, D)),
        _wspec((D, D)),
        _wspec((1, D)),
        _wspec((1, D)),
        pl.BlockSpec((1, 1, BM), lambda i: (i, 0, 0)),
    ],
    out_specs=[_bspec(BM, D)] + _PREP_OUT_SPECS,
    out_shape=[jax.ShapeDtypeStruct((N_PAD, D), f32)] + _PREP_OUTS,
)


def _post_prep_body(*refs):
    (e8_ref, gat_ref, den_ref, deg_ref, weT_ref, wfT_ref, bf_ref, bg_ref,
     res_ref, g_ref, bn_ref, wg_ref, asrc_ref, adst_ref,
     h_ref, hecat_ref, xlcat_ref, als_ref, ald_ref) = refs
    h2 = _post_block(e8_ref, gat_ref, den_ref, deg_ref, weT_ref, wfT_ref,
                     bf_ref, bg_ref, res_ref, g_ref, bn_ref)
    h_ref[...] = h2
    _prep_block(h2, deg_ref, wg_ref, asrc_ref, adst_ref,
                hecat_ref, xlcat_ref, als_ref, ald_ref)


_post_prep = pl.pallas_call(
    _post_prep_body,
    grid=(NBLK,),
    in_specs=_POST_IN_SPECS + [_wspec((D, D)), _wspec((1, D)), _wspec((1, D))],
    out_specs=[_bspec(BM, D)] + _PREP_OUT_SPECS,
    out_shape=[jax.ShapeDtypeStruct((N_PAD, D), f32)] + _PREP_OUTS,
)


def _post_read_body(*refs):
    (e8_ref, gat_ref, den_ref, deg_ref, weT_ref, wfT_ref, bf_ref, bg_ref,
     res_ref, g_ref, bn_ref, wr1T_ref, br1_ref, wr2_ref, br2_ref,
     out_ref) = refs
    h2 = _post_block(e8_ref, gat_ref, den_ref, deg_ref, weT_ref, wfT_ref,
                     bf_ref, bg_ref, res_ref, g_ref, bn_ref)
    r = jnp.maximum(
        jnp.dot(h2, wr1T_ref[...], preferred_element_type=f32)
        + br1_ref[...], 0.0)
    o = (r * wr2_ref[...]).sum(-1, keepdims=True) + br2_ref[...]
    out_ref[...] = 1.0 / (1.0 + jnp.exp(-o))


_post_read = pl.pallas_call(
    _post_read_body,
    grid=(NBLK,),
    in_specs=_POST_IN_SPECS + [
        _wspec((D, D // 2)),
        _wspec((1, D // 2)),
        _wspec((1, D // 2)),
        _wspec((1, 1)),
    ],
    out_specs=_bspec(BM, 1),
    out_shape=jax.ShapeDtypeStruct((N_PAD, 1), f32),
)


def _pad_idx(idx, tot):
    return jnp.concatenate(
        [idx, jnp.full((tot - idx.shape[0],), DUMMY, idx.dtype)])


def kernel(x, edge_index, Wemb, bemb, We1, Wg1, asrc1, adst1, bg1, Wf1, bf1,
           g1, bn1, We2, Wg2, asrc2, adst2, bg2, Wf2, bf2, g2, bn2, Wr1, br1,
           Wr2, br2):
    ei = edge_index.astype(i32)
    row, col = ei[0], ei[1]
    loop = jnp.arange(N, dtype=i32)
    g8 = _pad_idx(col, E8P).reshape(NTILE, NC8, 1, B)
    s8 = _pad_idx(row, E8P).reshape(NTILE, NC8, 1, B)
    e8i = jnp.concatenate([g8, s8], axis=2)
    sg = _pad_idx(jnp.concatenate([row, loop]), EGP).reshape(NTILE, NCG, 1, B)
    dg = _pad_idx(jnp.concatenate([col, loop]), EGP).reshape(NTILE, NCG, 1, B)
    gati = jnp.concatenate([sg, dg], axis=2)

    x_p = jnp.pad(x, ((0, N_PAD - N), (0, 0)))

    r2 = lambda a: a.reshape(1, -1)

    deg = _deg_kernel(g8.reshape(NTILE, NC8, B)).reshape(NBLK, 1, BM)

    h1, hecat, xlcat, als, ald = _prep_emb(
        x_p, Wemb.T, r2(bemb), Wg1, asrc1.reshape(1, D),
        adst1.reshape(1, D), deg)
    e8o, gato, den = _layer_sc(
        e8i, gati, hecat.reshape(NSC * N_PAD, 128),
        xlcat.reshape(NSC * N_PAD, 128),
        als.reshape(N_PAD * H), ald.reshape(N_PAD * H))
    h2, hecat, xlcat, als, ald = _post_prep(
        e8o.reshape(NSC, N_PAD, 128), gato.reshape(NSC, N_PAD, 128),
        den.reshape(N_PAD, H), deg, We1.T, Wf1.T, r2(bf1), r2(bg1), h1,
        r2(g1), r2(bn1), Wg2, asrc2.reshape(1, D), adst2.reshape(1, D))
    e8o, gato, den = _layer_sc(
        e8i, gati, hecat.reshape(NSC * N_PAD, 128),
        xlcat.reshape(NSC * N_PAD, 128),
        als.reshape(N_PAD * H), ald.reshape(N_PAD * H))
    out = _post_read(
        e8o.reshape(NSC, N_PAD, 128), gato.reshape(NSC, N_PAD, 128),
        den.reshape(N_PAD, H), deg, We2.T, Wf2.T, r2(bf2), r2(bg2), h2,
        r2(g2), r2(bn2), Wr1.T, r2(br1), Wr2, br2.reshape(1, 1))
    return out[:N]
